# Initial kernel scaffold; baseline (speedup 1.0000x reference)
import jax
import jax.numpy as jnp
from jax.experimental import pallas as pl


def kernel(x_user, x_item, edge_index_u2i, edge_index_i2u, W_in_user, b_in_user, W_in_item, b_in_item, l0_u2i_Wsrc, l0_u2i_Wdst, l0_u2i_asrc, l0_u2i_adst, l0_u2i_b, l0_i2u_Wsrc, l0_i2u_Wdst, l0_i2u_asrc, l0_i2u_adst, l0_i2u_b, l1_u2i_Wsrc, l1_u2i_Wdst, l1_u2i_asrc, l1_u2i_adst, l1_u2i_b, l1_i2u_Wsrc, l1_i2u_Wdst, l1_i2u_asrc, l1_i2u_adst, l1_i2u_b):
    raise NotImplementedError("write your pallas kernel here")


# speedup vs baseline: 19.7185x; 19.7185x over previous
import functools

import jax
import jax.numpy as jnp
from jax import lax
from jax.experimental import pallas as pl
from jax.experimental.pallas import tpu as pltpu
from jax.experimental.pallas import tpu_sc as plsc

N = 10000
E = 320000
RT = 10016
RO = 10240
NEG = -1.0e30

f32 = jnp.float32
i32 = jnp.int32


def _tc_in_proj(x, W1, b1, Wbig):
    B = 400
    K2 = Wbig.shape[1]

    def body(x_r, w1_r, b1_r, wb_r, o_r):
        h = jnp.dot(x_r[...], w1_r[...], preferred_element_type=f32) + b1_r[...]
        h = jnp.where(h > 0, h, jnp.exp(jnp.minimum(h, 0.0)) - 1.0)
        o_r[...] = jnp.dot(h, wb_r[...], preferred_element_type=f32)

    return pl.pallas_call(
        body,
        grid=(N // B,),
        in_specs=[
            pl.BlockSpec((B, x.shape[1]), lambda i: (i, 0)),
            pl.BlockSpec(W1.shape, lambda i: (0, 0)),
            pl.BlockSpec(b1.shape, lambda i: (0, 0)),
            pl.BlockSpec(Wbig.shape, lambda i: (0, 0)),
        ],
        out_specs=pl.BlockSpec((B, K2), lambda i: (i, 0)),
        out_shape=jax.ShapeDtypeStruct((N, K2), f32),
    )(x, W1, b1, Wbig)


def _tc_mid(num, den, b, Wbig):
    B = 400
    K2 = Wbig.shape[1]

    def body(n_r, d_r, b_r, wb_r, o_r):
        d = d_r[...]
        drep = jnp.concatenate(
            [jnp.broadcast_to(d[:, h:h + 1], (B, 64)) for h in range(4)], axis=1)
        t = n_r[...] / (drep + 1e-16) + b_r[...]
        t = jnp.where(t > 0, t, jnp.exp(jnp.minimum(t, 0.0)) - 1.0)
        o_r[...] = jnp.dot(t, wb_r[...], preferred_element_type=f32)

    return pl.pallas_call(
        body,
        grid=(N // B,),
        in_specs=[
            pl.BlockSpec((B, 256), lambda i: (i, 0)),
            pl.BlockSpec((B, 4), lambda i: (i, 0)),
            pl.BlockSpec(b.shape, lambda i: (0, 0)),
            pl.BlockSpec(Wbig.shape, lambda i: (0, 0)),
        ],
        out_specs=pl.BlockSpec((B, K2), lambda i: (i, 0)),
        out_shape=jax.ShapeDtypeStruct((N, K2), f32),
    )(num, den, b, Wbig)


def _tc_final(t0, t1, b):
    B = 1000

    def body(a_r, c_r, b_r, o_r):
        s = a_r[...] + c_r[...]
        o_r[...] = s[:, :32] / (s[:, 32:33] + 1e-16) + b_r[...]

    return pl.pallas_call(
        body,
        grid=(N // B,),
        in_specs=[
            pl.BlockSpec((B, 33), lambda i: (i, 0)),
            pl.BlockSpec((B, 33), lambda i: (i, 0)),
            pl.BlockSpec(b.shape, lambda i: (0, 0)),
        ],
        out_specs=pl.BlockSpec((B, 32), lambda i: (i, 0)),
        out_shape=jax.ShapeDtypeStruct((N, 32), f32),
    )(t0, t1, b)


_GDN = None


def _bcast_lane(ev, i):
    gdn = lax.GatherDimensionNumbers(
        offset_dims=(), collapsed_slice_dims=(0,), start_index_map=(0,))
    sel = jnp.full((16, 1), i, i32)
    return lax.gather(ev, sel, dimension_numbers=gdn, slice_sizes=(1,),
                      mode=lax.GatherScatterMode.PROMISE_IN_BOUNDS)


def _make_sc_scores(EP, HS):
    CPT = EP // 16 // 128
    mesh = plsc.VectorSubcoreMesh(core_axis_name="c", subcore_axis_name="s")

    @functools.partial(
        pl.kernel,
        out_type=jax.ShapeDtypeStruct((2, EP * HS), f32),
        mesh=mesh,
        compiler_params=pltpu.CompilerParams(
            needs_layout_passes=False, use_tc_tiling_on_sc=False),
        scratch_types=[
            pltpu.VMEM((RT * HS,), f32),
            pltpu.VMEM((RT * HS,), f32),
            pltpu.VMEM((128,), i32),
            pltpu.VMEM((128,), i32),
            pltpu.VMEM((128 * HS,), f32),
        ],
    )
    def scores(as_hbm, ad_hbm, src_hbm, dst_hbm, ex_hbm,
               as_v, ad_v, src_v, dst_v, ex_v):
        c = lax.axis_index("c")
        s = lax.axis_index("s")
        pltpu.sync_copy(as_hbm.at[c], as_v)
        pltpu.sync_copy(ad_hbm.at[c], ad_v)

        def chunk(g, _):
            base = s * (EP // 16) + g * 128
            pltpu.sync_copy(src_hbm.at[c, pl.ds(base, 128)], src_v)
            pltpu.sync_copy(dst_hbm.at[c, pl.ds(base, 128)], dst_v)

            def grp(q, _):
                sv = src_v[pl.ds(q * 16, 16)]
                dv = dst_v[pl.ds(q * 16, 16)]
                for h in range(HS):
                    a = (plsc.load_gather(as_v, [sv * HS + h]) +
                         plsc.load_gather(ad_v, [dv * HS + h]))
                    a = jnp.where(a > 0, a, 0.2 * a)
                    ex_v[pl.ds(h * 128 + q * 16, 16)] = jnp.exp(a)
                return 0

            lax.fori_loop(0, 8, grp, 0)
            pltpu.sync_copy(ex_v, ex_hbm.at[c, pl.ds(base * HS, 128 * HS)])
            return 0

        lax.fori_loop(0, CPT, chunk, 0)

    return scores


def _make_sc_msg(EP, HS, F):
    W = F + 16
    CH = F // HS
    CPT = EP // 16 // 128
    mesh = plsc.VectorSubcoreMesh(core_axis_name="c", subcore_axis_name="s")

    @functools.partial(
        pl.kernel,
        out_type=jax.ShapeDtypeStruct((2, RO, W), f32),
        mesh=mesh,
        compiler_params=pltpu.CompilerParams(
            needs_layout_passes=False, use_tc_tiling_on_sc=False),
        scratch_types=[
            pltpu.VMEM((128,), i32),
            pltpu.VMEM((128,), i32),
            pltpu.VMEM((128,), i32),
            pltpu.VMEM((128 * HS,), f32),
            pltpu.VMEM((128, F), f32),
            pltpu.VMEM((128, W), f32),
            pltpu.VMEM_SHARED((RO, W), f32),
            pltpu.SemaphoreType.DMA,
        ],
    )
    def msg(hs_hbm, ex_hbm, src_hbm, dst_hbm, out_hbm,
            src_v, dst_v, srcg_v, ex_v, rows, orow, out_sh, sem):
        c = lax.axis_index("c")
        s = lax.axis_index("s")

        zero16 = jnp.zeros((16,), f32)

        def zero_row(j, _):
            for t in range(W // 16):
                orow[j, pl.ds(t * 16, 16)] = zero16
            return 0

        lax.fori_loop(0, 128, zero_row, 0)
        for t in range(5):
            pltpu.sync_copy(orow, out_sh.at[pl.ds(s * 640 + t * 128, 128)])
        plsc.subcore_barrier()

        offv = lax.broadcast(c * RT, (16,)).astype(i32)
        lane = lax.iota(i32, 16)

        def chunk(g, _):
            base = s * (EP // 16) + g * 128
            pltpu.sync_copy(src_hbm.at[c, pl.ds(base, 128)], src_v)
            pltpu.sync_copy(dst_hbm.at[c, pl.ds(base, 128)], dst_v)
            pltpu.sync_copy(ex_hbm.at[c, pl.ds(base * HS, 128 * HS)], ex_v)

            def adj(q, _):
                srcg_v[pl.ds(q * 16, 16)] = src_v[pl.ds(q * 16, 16)] + offv
                return 0

            lax.fori_loop(0, 8, adj, 0)
            pltpu.async_copy(hs_hbm.at[srcg_v], rows, sem).wait()

            def grp(q, _):
                evs = [ex_v[pl.ds(h * 128 + q * 16, 16)] for h in range(HS)]
                for i in range(16):
                    e = q * 16 + i
                    bexs = [_bcast_lane(ev, i) for ev in evs]
                    for t in range(F // 16):
                        h = (t * 16) // CH
                        orow[e, pl.ds(t * 16, 16)] = (
                            rows[e, pl.ds(t * 16, 16)] * bexs[h])
                    tail = zero16
                    for h in range(HS):
                        tail = jnp.where(lane == h, bexs[h], tail)
                    orow[e, pl.ds(F, 16)] = tail
                return 0

            lax.fori_loop(0, 8, grp, 0)
            pltpu.sync_copy(orow, out_sh.at[dst_v], add=True)
            return 0

        lax.fori_loop(0, CPT, chunk, 0)
        plsc.subcore_barrier()
        for t in range(5):
            pltpu.sync_copy(out_sh.at[pl.ds(s * 640 + t * 128, 128)], orow)
            pltpu.sync_copy(orow, out_hbm.at[c, pl.ds(s * 640 + t * 128, 128)])

    return msg


_sc_scores_l0 = _make_sc_scores(EP=327680, HS=2)
_sc_scores_l1 = _make_sc_scores(EP=163840, HS=1)
_sc_msg_l0 = _make_sc_msg(EP=327680, HS=2, F=128)
_sc_msg_l1 = _make_sc_msg(EP=163840, HS=1, F=32)


def _sc_conv_l0(hs_t, as_t, ad_t, src, dst):
    ex = _sc_scores_l0(as_t, ad_t, src, dst)
    return _sc_msg_l0(hs_t, ex, src, dst)


def _sc_conv_l1(hs_t, as_t, ad_t, src, dst):
    ex = _sc_scores_l1(as_t, ad_t, src, dst)
    return _sc_msg_l1(hs_t, ex, src, dst)


def _pad_rows(x, val=0.0):
    return jnp.concatenate(
        [x, jnp.full((RT - N,) + x.shape[1:], val, x.dtype)], axis=0)


def _prep_edges_l0(ei):
    src = ei[0].astype(i32)
    dst = ei[1].astype(i32)
    pad = jnp.full((327680 - E,), N, i32)
    srcp = jnp.concatenate([src, pad])
    dstp = jnp.concatenate([dst, pad])
    return (jnp.broadcast_to(srcp[None], (2, 327680)),
            jnp.broadcast_to(dstp[None], (2, 327680)))


def _prep_edges_l1(ei):
    src = ei[0].astype(i32)
    dst = ei[1].astype(i32)
    half = E // 2
    pad = jnp.full((163840 - half,), N, i32)
    srcp = jnp.stack([jnp.concatenate([src[:half], pad]),
                      jnp.concatenate([src[half:], pad])])
    dstp = jnp.stack([jnp.concatenate([dst[:half], pad]),
                      jnp.concatenate([dst[half:], pad])])
    return srcp, dstp


def _score_tables_l0(a):
    ap = _pad_rows(a, NEG)
    return ap.reshape(RT, 2, 2).transpose(1, 0, 2).reshape(2, RT * 2)


def _score_tables_l1(a):
    ap = _pad_rows(a[:, None], NEG).reshape(RT)
    return jnp.broadcast_to(ap[None], (2, RT))


def _hs_table_l0(hs):
    hp = _pad_rows(hs)
    return jnp.concatenate([hp[:, :128], hp[:, 128:]], axis=0)


def _hs_table_l1(hs):
    hp = _pad_rows(hs)
    return jnp.concatenate([hp, hp], axis=0)


def _fold_l0(Wsrc, avec):
    return jnp.einsum('khc,hc->kh', Wsrc.reshape(64, 4, 64), avec)


def kernel(x_user, x_item, edge_index_u2i, edge_index_i2u,
           W_in_user, b_in_user, W_in_item, b_in_item,
           l0_u2i_Wsrc, l0_u2i_Wdst, l0_u2i_asrc, l0_u2i_adst, l0_u2i_b,
           l0_i2u_Wsrc, l0_i2u_Wdst, l0_i2u_asrc, l0_i2u_adst, l0_i2u_b,
           l1_u2i_Wsrc, l1_u2i_Wdst, l1_u2i_asrc, l1_u2i_adst, l1_u2i_b,
           l1_i2u_Wsrc, l1_i2u_Wdst, l1_i2u_asrc, l1_i2u_adst, l1_i2u_b):
    ws0_u2i = _fold_l0(l0_u2i_Wsrc, l0_u2i_asrc)
    wd0_u2i = _fold_l0(l0_u2i_Wdst, l0_u2i_adst)
    ws0_i2u = _fold_l0(l0_i2u_Wsrc, l0_i2u_asrc)
    wd0_i2u = _fold_l0(l0_i2u_Wdst, l0_i2u_adst)
    WbigU = jnp.concatenate([l0_u2i_Wsrc, ws0_u2i, wd0_i2u], axis=1)
    WbigI = jnp.concatenate([l0_i2u_Wsrc, ws0_i2u, wd0_u2i], axis=1)

    bigU = _tc_in_proj(x_user, W_in_user, b_in_user[None], WbigU)
    bigI = _tc_in_proj(x_item, W_in_item, b_in_item[None], WbigI)

    su0, du0 = _prep_edges_l0(edge_index_u2i)
    si0, di0 = _prep_edges_l0(edge_index_i2u)

    outU = _sc_conv_l0(_hs_table_l0(bigU[:, :256]),
                       _score_tables_l0(bigU[:, 256:260]),
                       _score_tables_l0(bigI[:, 260:264]),
                       su0, du0)
    outI = _sc_conv_l0(_hs_table_l0(bigI[:, :256]),
                       _score_tables_l0(bigI[:, 256:260]),
                       _score_tables_l0(bigU[:, 260:264]),
                       si0, di0)

    num_i = jnp.concatenate([outU[0, :N, :128], outU[1, :N, :128]], axis=1)
    den_i = jnp.concatenate([outU[0, :N, 128:130], outU[1, :N, 128:130]], axis=1)
    num_u = jnp.concatenate([outI[0, :N, :128], outI[1, :N, :128]], axis=1)
    den_u = jnp.concatenate([outI[0, :N, 128:130], outI[1, :N, 128:130]], axis=1)

    ws1_u2i = l1_u2i_Wsrc @ l1_u2i_asrc[0]
    wd1_u2i = l1_u2i_Wdst @ l1_u2i_adst[0]
    ws1_i2u = l1_i2u_Wsrc @ l1_i2u_asrc[0]
    wd1_i2u = l1_i2u_Wdst @ l1_i2u_adst[0]
    W1bigU = jnp.concatenate(
        [l1_u2i_Wsrc, ws1_u2i[:, None], wd1_i2u[:, None]], axis=1)
    W1bigI = jnp.concatenate(
        [l1_i2u_Wsrc, ws1_i2u[:, None], wd1_u2i[:, None]], axis=1)

    big1U = _tc_mid(num_u, den_u, l0_i2u_b[None], W1bigU)
    big1I = _tc_mid(num_i, den_i, l0_u2i_b[None], W1bigI)

    su1, du1 = _prep_edges_l1(edge_index_u2i)
    si1, di1 = _prep_edges_l1(edge_index_i2u)

    out1U = _sc_conv_l1(_hs_table_l1(big1U[:, :32]),
                        _score_tables_l1(big1U[:, 32]),
                        _score_tables_l1(big1I[:, 33]),
                        su1, du1)
    out1I = _sc_conv_l1(_hs_table_l1(big1I[:, :32]),
                        _score_tables_l1(big1I[:, 32]),
                        _score_tables_l1(big1U[:, 33]),
                        si1, di1)

    hi2 = _tc_final(out1U[0, :N, :33], out1U[1, :N, :33], l1_u2i_b[None])
    hu2 = _tc_final(out1I[0, :N, :33], out1I[1, :N, :33], l1_i2u_b[None])
    return hu2, hi2


# speedup vs baseline: 23.6873x; 1.2013x over previous
import functools

import jax
import jax.numpy as jnp
from jax import lax
from jax.experimental import pallas as pl
from jax.experimental.pallas import tpu as pltpu
from jax.experimental.pallas import tpu_sc as plsc

N = 10000
E = 320000
RT = 10016
RO = 10016
NEG = -1.0e30

f32 = jnp.float32
i32 = jnp.int32

_SC_PARAMS = pltpu.CompilerParams(
    needs_layout_passes=False, use_tc_tiling_on_sc=False)


def _tc_in_proj(x, W1, b1, Wbig):
    B = 400
    K2 = Wbig.shape[1]

    def body(x_r, w1_r, b1_r, wb_r, o_r):
        h = jnp.dot(x_r[...], w1_r[...], preferred_element_type=f32) + b1_r[...]
        h = jnp.where(h > 0, h, jnp.exp(jnp.minimum(h, 0.0)) - 1.0)
        o_r[...] = jnp.dot(h, wb_r[...], preferred_element_type=f32)

    return pl.pallas_call(
        body,
        grid=(N // B,),
        in_specs=[
            pl.BlockSpec((B, x.shape[1]), lambda i: (i, 0)),
            pl.BlockSpec(W1.shape, lambda i: (0, 0)),
            pl.BlockSpec(b1.shape, lambda i: (0, 0)),
            pl.BlockSpec(Wbig.shape, lambda i: (0, 0)),
        ],
        out_specs=pl.BlockSpec((B, K2), lambda i: (i, 0)),
        out_shape=jax.ShapeDtypeStruct((N, K2), f32),
    )(x, W1, b1, Wbig)


def _tc_mid(num, den, b, Wbig):
    B = 400
    K2 = Wbig.shape[1]

    def body(n_r, d_r, b_r, wb_r, o_r):
        d = d_r[...]
        drep = jnp.concatenate(
            [jnp.broadcast_to(d[:, h:h + 1], (B, 64)) for h in range(4)], axis=1)
        t = n_r[...] / (drep + 1e-16) + b_r[...]
        t = jnp.where(t > 0, t, jnp.exp(jnp.minimum(t, 0.0)) - 1.0)
        o_r[...] = jnp.dot(t, wb_r[...], preferred_element_type=f32)

    return pl.pallas_call(
        body,
        grid=(N // B,),
        in_specs=[
            pl.BlockSpec((B, 256), lambda i: (i, 0)),
            pl.BlockSpec((B, 4), lambda i: (i, 0)),
            pl.BlockSpec(b.shape, lambda i: (0, 0)),
            pl.BlockSpec(Wbig.shape, lambda i: (0, 0)),
        ],
        out_specs=pl.BlockSpec((B, K2), lambda i: (i, 0)),
        out_shape=jax.ShapeDtypeStruct((N, K2), f32),
    )(num, den, b, Wbig)


def _tc_final(t0, t1, b):
    B = 1000

    def body(a_r, c_r, b_r, o_r):
        s = a_r[...] + c_r[...]
        o_r[...] = s[:, :32] / (s[:, 32:33] + 1e-16) + b_r[...]

    return pl.pallas_call(
        body,
        grid=(N // B,),
        in_specs=[
            pl.BlockSpec((B, 33), lambda i: (i, 0)),
            pl.BlockSpec((B, 33), lambda i: (i, 0)),
            pl.BlockSpec(b.shape, lambda i: (0, 0)),
        ],
        out_specs=pl.BlockSpec((B, 32), lambda i: (i, 0)),
        out_shape=jax.ShapeDtypeStruct((N, 32), f32),
    )(t0, t1, b)


_GDN = lax.GatherDimensionNumbers(
    offset_dims=(), collapsed_slice_dims=(0,), start_index_map=(0,))


def _bcast_lane(ev, lanei):
    sel = lax.broadcast(lanei, (16,)).astype(i32).reshape(16, 1)
    return lax.gather(ev, sel, dimension_numbers=_GDN, slice_sizes=(1,),
                      mode=lax.GatherScatterMode.PROMISE_IN_BOUNDS)


def _make_sc_scores(EP, HS):
    EPT = EP // 16
    SUP = EPT // 1024
    mesh = plsc.VectorSubcoreMesh(core_axis_name="c", subcore_axis_name="s")

    @functools.partial(
        pl.kernel,
        out_type=jax.ShapeDtypeStruct((2, EP // 64, 64 * HS), f32),
        mesh=mesh,
        compiler_params=_SC_PARAMS,
        scratch_types=[
            pltpu.VMEM((RT * HS,), f32),
            pltpu.VMEM((RT * HS,), f32),
            pltpu.VMEM((16, 64), i32), pltpu.VMEM((16, 64), i32),
            pltpu.VMEM((16, 64), i32), pltpu.VMEM((16, 64), i32),
            pltpu.VMEM((16, 64 * HS), f32), pltpu.VMEM((16, 64 * HS), f32),
            pltpu.SemaphoreType.DMA, pltpu.SemaphoreType.DMA,
            pltpu.SemaphoreType.DMA, pltpu.SemaphoreType.DMA,
        ],
    )
    def scores(as_hbm, ad_hbm, src_hbm, dst_hbm, ex_hbm,
               as_v, ad_v, srcA, srcB, dstA, dstB, exA, exB,
               lsemA, lsemB, ssemA, ssemB):
        c = lax.axis_index("c")
        s = lax.axis_index("s")
        pltpu.sync_copy(as_hbm.at[c], as_v)
        pltpu.sync_copy(ad_hbm.at[c], ad_v)

        srcs = [srcA, srcB]
        dsts = [dstA, dstB]
        exs = [exA, exB]
        lsems = [lsemA, lsemB]
        ssems = [ssemA, ssemB]

        def issue_loads(g):
            b = g % 2
            base64 = s * (EPT // 64) + g * 16
            d1 = pltpu.async_copy(src_hbm.at[c, pl.ds(base64, 16)],
                                  srcs[b], lsems[b])
            d2 = pltpu.async_copy(dst_hbm.at[c, pl.ds(base64, 16)],
                                  dsts[b], lsems[b])
            return (d1, d2)

        lds = {0: issue_loads(0)}
        sts = {}
        for g in range(SUP):
            b = g % 2
            if g + 1 < SUP:
                lds[g + 1] = issue_loads(g + 1)
            lds[g][0].wait()
            lds[g][1].wait()
            if g >= 2:
                sts[g - 2].wait()
            src_v, dst_v, ex_v = srcs[b], dsts[b], exs[b]

            def row(r, _):
                for t in range(4):
                    sv = src_v[r, pl.ds(t * 16, 16)]
                    dv = dst_v[r, pl.ds(t * 16, 16)]
                    for h in range(HS):
                        a = (plsc.load_gather(as_v, [sv * HS + h]) +
                             plsc.load_gather(ad_v, [dv * HS + h]))
                        a = jnp.where(a > 0, a, 0.2 * a)
                        ex_v[r, pl.ds(h * 64 + t * 16, 16)] = jnp.exp(a)
                return 0

            lax.fori_loop(0, 16, row, 0)
            base64 = s * (EPT // 64) + g * 16
            sts[g] = pltpu.async_copy(
                ex_v, ex_hbm.at[c, pl.ds(base64, 16)], ssems[b])
        sts[SUP - 2].wait()
        sts[SUP - 1].wait()

    return scores


def _make_sc_msg(EP, HS, F):
    W = F + 16
    CH = F // HS
    EPT = EP // 16
    SUP = EPT // 1024
    RPT = RO // 16
    mesh = plsc.VectorSubcoreMesh(core_axis_name="c", subcore_axis_name="s")

    @functools.partial(
        pl.kernel,
        out_type=jax.ShapeDtypeStruct((2, RO, W), f32),
        mesh=mesh,
        compiler_params=_SC_PARAMS,
        scratch_types=[
            pltpu.VMEM((16, 64), i32),
            pltpu.VMEM((16, 64), i32),
            pltpu.VMEM((16, 64), i32),
            pltpu.VMEM((16, 64 * HS), f32),
            pltpu.VMEM((64, F), f32), pltpu.VMEM((64, F), f32),
            pltpu.VMEM((64, W), f32), pltpu.VMEM((64, W), f32),
            pltpu.VMEM_SHARED((RO, W), f32),
            pltpu.SemaphoreType.DMA,
            pltpu.SemaphoreType.DMA, pltpu.SemaphoreType.DMA,
            pltpu.SemaphoreType.DMA, pltpu.SemaphoreType.DMA,
        ],
    )
    def msg(hs_hbm, ex_hbm, src_hbm, dst_hbm, out_hbm,
            src_v, dst_v, srcg_v, ex_v, rowsA, rowsB, orowA, orowB,
            out_sh, lsem, gsemA, gsemB, ssemA, ssemB):
        c = lax.axis_index("c")
        s = lax.axis_index("s")

        zero16 = jnp.zeros((16,), f32)
        rows = [rowsA, rowsB]
        orows = [orowA, orowB]
        gsems = [gsemA, gsemB]
        ssems = [ssemA, ssemB]

        def zero_row(j, _):
            for t in range(W // 16):
                orowA[j, pl.ds(t * 16, 16)] = zero16
            return 0

        lax.fori_loop(0, 64, zero_row, 0)
        r0 = s * RPT
        for t in range(9):
            pltpu.sync_copy(orowA, out_sh.at[pl.ds(r0 + t * 64, 64)])
        pltpu.sync_copy(orowA.at[pl.ds(0, RPT - 576)],
                        out_sh.at[pl.ds(r0 + 576, RPT - 576)])
        plsc.subcore_barrier()

        offv = lax.broadcast(c * RT, (16,)).astype(i32)
        lane = lax.iota(i32, 16)

        def sup(g, _):
            base64 = s * (EPT // 64) + g * 16
            d1 = pltpu.async_copy(src_hbm.at[c, pl.ds(base64, 16)],
                                  src_v, lsem)
            d2 = pltpu.async_copy(dst_hbm.at[c, pl.ds(base64, 16)],
                                  dst_v, lsem)
            d3 = pltpu.async_copy(ex_hbm.at[c, pl.ds(base64, 16)],
                                  ex_v, lsem)
            d1.wait(); d2.wait(); d3.wait()

            def adj(r, _):
                for t in range(4):
                    srcg_v[r, pl.ds(t * 16, 16)] = (
                        src_v[r, pl.ds(t * 16, 16)] + offv)
                return 0

            lax.fori_loop(0, 16, adj, 0)

            for j in range(16):
                b = j % 2
                pltpu.async_copy(hs_hbm.at[srcg_v.at[j]],
                                 rows[b], gsems[b]).wait()
                rb = rows[b]
                ob = orows[b]

                def quad(q, _):
                    g16 = (q // 4) * 16
                    evs = [ex_v[j, pl.ds(h * 64 + g16, 16)]
                           for h in range(HS)]
                    for k in range(4):
                        lanei = (q % 4) * 4 + k
                        bexs = [_bcast_lane(ev, lanei) for ev in evs]
                        e = q * 4 + k
                        for t in range(F // 16):
                            h = (t * 16) // CH
                            ob[e, pl.ds(t * 16, 16)] = (
                                rb[e, pl.ds(t * 16, 16)] * bexs[h])
                        tail = zero16
                        for h in range(HS):
                            tail = jnp.where(lane == h, bexs[h], tail)
                        ob[e, pl.ds(F, 16)] = tail
                    return 0

                lax.fori_loop(0, 16, quad, 0)
                pltpu.sync_copy(ob, out_sh.at[dst_v.at[j]], add=True)
            return 0

        lax.fori_loop(0, SUP, sup, 0)
        plsc.subcore_barrier()
        for t in range(9):
            pltpu.sync_copy(out_sh.at[pl.ds(r0 + t * 64, 64)], orowA)
            pltpu.sync_copy(orowA, out_hbm.at[c, pl.ds(r0 + t * 64, 64)])
        pltpu.sync_copy(out_sh.at[pl.ds(r0 + 576, RPT - 576)],
                        orowA.at[pl.ds(0, RPT - 576)])
        pltpu.sync_copy(orowA.at[pl.ds(0, RPT - 576)],
                        out_hbm.at[c, pl.ds(r0 + 576, RPT - 576)])

    return msg


_sc_scores_l0 = _make_sc_scores(EP=327680, HS=2)
_sc_scores_l1 = _make_sc_scores(EP=163840, HS=1)
_sc_msg_l0 = _make_sc_msg(EP=327680, HS=2, F=128)
_sc_msg_l1 = _make_sc_msg(EP=163840, HS=1, F=32)


def _sc_conv_l0(hs_t, as_t, ad_t, src, dst):
    ex = _sc_scores_l0(as_t, ad_t, src, dst)
    return _sc_msg_l0(hs_t, ex, src, dst)


def _sc_conv_l1(hs_t, as_t, ad_t, src, dst):
    ex = _sc_scores_l1(as_t, ad_t, src, dst)
    return _sc_msg_l1(hs_t, ex, src, dst)


def _pad_rows(x, val=0.0):
    return jnp.concatenate(
        [x, jnp.full((RT - N,) + x.shape[1:], val, x.dtype)], axis=0)


def _prep_edges_l0(ei):
    src = ei[0].astype(i32)
    dst = ei[1].astype(i32)
    pad = jnp.full((327680 - E,), N, i32)
    srcp = jnp.concatenate([src, pad]).reshape(327680 // 64, 64)
    dstp = jnp.concatenate([dst, pad]).reshape(327680 // 64, 64)
    return (jnp.broadcast_to(srcp[None], (2, 327680 // 64, 64)),
            jnp.broadcast_to(dstp[None], (2, 327680 // 64, 64)))


def _prep_edges_l1(ei):
    src = ei[0].astype(i32)
    dst = ei[1].astype(i32)
    half = E // 2
    pad = jnp.full((163840 - half,), N, i32)
    srcp = jnp.stack([jnp.concatenate([src[:half], pad]),
                      jnp.concatenate([src[half:], pad])])
    dstp = jnp.stack([jnp.concatenate([dst[:half], pad]),
                      jnp.concatenate([dst[half:], pad])])
    return (srcp.reshape(2, 163840 // 64, 64),
            dstp.reshape(2, 163840 // 64, 64))


def _score_tables_l0(a):
    ap = _pad_rows(a, NEG)
    return ap.reshape(RT, 2, 2).transpose(1, 0, 2).reshape(2, RT * 2)


def _score_tables_l1(a):
    ap = _pad_rows(a[:, None], NEG).reshape(RT)
    return jnp.broadcast_to(ap[None], (2, RT))


def _hs_table_l0(hs):
    hp = _pad_rows(hs)
    return jnp.concatenate([hp[:, :128], hp[:, 128:]], axis=0)


def _hs_table_l1(hs):
    hp = _pad_rows(hs)
    return jnp.concatenate([hp, hp], axis=0)


def _fold_l0(Wsrc, avec):
    return jnp.einsum('khc,hc->kh', Wsrc.reshape(64, 4, 64), avec)


def kernel(x_user, x_item, edge_index_u2i, edge_index_i2u,
           W_in_user, b_in_user, W_in_item, b_in_item,
           l0_u2i_Wsrc, l0_u2i_Wdst, l0_u2i_asrc, l0_u2i_adst, l0_u2i_b,
           l0_i2u_Wsrc, l0_i2u_Wdst, l0_i2u_asrc, l0_i2u_adst, l0_i2u_b,
           l1_u2i_Wsrc, l1_u2i_Wdst, l1_u2i_asrc, l1_u2i_adst, l1_u2i_b,
           l1_i2u_Wsrc, l1_i2u_Wdst, l1_i2u_asrc, l1_i2u_adst, l1_i2u_b):
    ws0_u2i = _fold_l0(l0_u2i_Wsrc, l0_u2i_asrc)
    wd0_u2i = _fold_l0(l0_u2i_Wdst, l0_u2i_adst)
    ws0_i2u = _fold_l0(l0_i2u_Wsrc, l0_i2u_asrc)
    wd0_i2u = _fold_l0(l0_i2u_Wdst, l0_i2u_adst)
    WbigU = jnp.concatenate([l0_u2i_Wsrc, ws0_u2i, wd0_i2u], axis=1)
    WbigI = jnp.concatenate([l0_i2u_Wsrc, ws0_i2u, wd0_u2i], axis=1)

    bigU = _tc_in_proj(x_user, W_in_user, b_in_user[None], WbigU)
    bigI = _tc_in_proj(x_item, W_in_item, b_in_item[None], WbigI)

    su0, du0 = _prep_edges_l0(edge_index_u2i)
    si0, di0 = _prep_edges_l0(edge_index_i2u)

    outU = _sc_conv_l0(_hs_table_l0(bigU[:, :256]),
                       _score_tables_l0(bigU[:, 256:260]),
                       _score_tables_l0(bigI[:, 260:264]),
                       su0, du0)
    outI = _sc_conv_l0(_hs_table_l0(bigI[:, :256]),
                       _score_tables_l0(bigI[:, 256:260]),
                       _score_tables_l0(bigU[:, 260:264]),
                       si0, di0)

    num_i = jnp.concatenate([outU[0, :N, :128], outU[1, :N, :128]], axis=1)
    den_i = jnp.concatenate([outU[0, :N, 128:130], outU[1, :N, 128:130]], axis=1)
    num_u = jnp.concatenate([outI[0, :N, :128], outI[1, :N, :128]], axis=1)
    den_u = jnp.concatenate([outI[0, :N, 128:130], outI[1, :N, 128:130]], axis=1)

    ws1_u2i = l1_u2i_Wsrc @ l1_u2i_asrc[0]
    wd1_u2i = l1_u2i_Wdst @ l1_u2i_adst[0]
    ws1_i2u = l1_i2u_Wsrc @ l1_i2u_asrc[0]
    wd1_i2u = l1_i2u_Wdst @ l1_i2u_adst[0]
    W1bigU = jnp.concatenate(
        [l1_u2i_Wsrc, ws1_u2i[:, None], wd1_i2u[:, None]], axis=1)
    W1bigI = jnp.concatenate(
        [l1_i2u_Wsrc, ws1_i2u[:, None], wd1_u2i[:, None]], axis=1)

    big1U = _tc_mid(num_u, den_u, l0_i2u_b[None], W1bigU)
    big1I = _tc_mid(num_i, den_i, l0_u2i_b[None], W1bigI)

    su1, du1 = _prep_edges_l1(edge_index_u2i)
    si1, di1 = _prep_edges_l1(edge_index_i2u)

    out1U = _sc_conv_l1(_hs_table_l1(big1U[:, :32]),
                        _score_tables_l1(big1U[:, 32]),
                        _score_tables_l1(big1I[:, 33]),
                        su1, du1)
    out1I = _sc_conv_l1(_hs_table_l1(big1I[:, :32]),
                        _score_tables_l1(big1I[:, 32]),
                        _score_tables_l1(big1U[:, 33]),
                        si1, di1)

    hi2 = _tc_final(out1U[0, :N, :33], out1U[1, :N, :33], l1_u2i_b[None])
    hu2 = _tc_final(out1I[0, :N, :33], out1I[1, :N, :33], l1_i2u_b[None])
    return hu2, hi2


# speedup vs baseline: 33.5865x; 1.4179x over previous
import functools

import jax
import jax.numpy as jnp
from jax import lax
from jax.experimental import pallas as pl
from jax.experimental.pallas import tpu as pltpu
from jax.experimental.pallas import tpu_sc as plsc

N = 10000
E = 320000
RT = 10016
RO = 10016
NEG = -1.0e30

f32 = jnp.float32
i32 = jnp.int32

_SC_PARAMS = pltpu.CompilerParams(
    needs_layout_passes=False, use_tc_tiling_on_sc=False)


def _tc_in_proj(x, W1, b1, Wbig):
    B = 400
    K2 = Wbig.shape[1]

    def body(x_r, w1_r, b1_r, wb_r, o_r):
        h = jnp.dot(x_r[...], w1_r[...], preferred_element_type=f32) + b1_r[...]
        h = jnp.where(h > 0, h, jnp.exp(jnp.minimum(h, 0.0)) - 1.0)
        o_r[...] = jnp.dot(h, wb_r[...], preferred_element_type=f32)

    return pl.pallas_call(
        body,
        grid=(N // B,),
        in_specs=[
            pl.BlockSpec((B, x.shape[1]), lambda i: (i, 0)),
            pl.BlockSpec(W1.shape, lambda i: (0, 0)),
            pl.BlockSpec(b1.shape, lambda i: (0, 0)),
            pl.BlockSpec(Wbig.shape, lambda i: (0, 0)),
        ],
        out_specs=pl.BlockSpec((B, K2), lambda i: (i, 0)),
        out_shape=jax.ShapeDtypeStruct((N, K2), f32),
    )(x, W1, b1, Wbig)


def _tc_mid(num, den, b, Wbig):
    B = 400
    K2 = Wbig.shape[1]

    def body(n_r, d_r, b_r, wb_r, o_r):
        d = d_r[...]
        drep = jnp.concatenate(
            [jnp.broadcast_to(d[:, h:h + 1], (B, 64)) for h in range(4)], axis=1)
        t = n_r[...] / (drep + 1e-16) + b_r[...]
        t = jnp.where(t > 0, t, jnp.exp(jnp.minimum(t, 0.0)) - 1.0)
        o_r[...] = jnp.dot(t, wb_r[...], preferred_element_type=f32)

    return pl.pallas_call(
        body,
        grid=(N // B,),
        in_specs=[
            pl.BlockSpec((B, 256), lambda i: (i, 0)),
            pl.BlockSpec((B, 4), lambda i: (i, 0)),
            pl.BlockSpec(b.shape, lambda i: (0, 0)),
            pl.BlockSpec(Wbig.shape, lambda i: (0, 0)),
        ],
        out_specs=pl.BlockSpec((B, K2), lambda i: (i, 0)),
        out_shape=jax.ShapeDtypeStruct((N, K2), f32),
    )(num, den, b, Wbig)


def _tc_final(t0, t1, b):
    B = 1000

    def body(a_r, c_r, b_r, o_r):
        s = a_r[...] + c_r[...]
        o_r[...] = s[:, :32] / (s[:, 32:33] + 1e-16) + b_r[...]

    return pl.pallas_call(
        body,
        grid=(N // B,),
        in_specs=[
            pl.BlockSpec((B, 33), lambda i: (i, 0)),
            pl.BlockSpec((B, 33), lambda i: (i, 0)),
            pl.BlockSpec(b.shape, lambda i: (0, 0)),
        ],
        out_specs=pl.BlockSpec((B, 32), lambda i: (i, 0)),
        out_shape=jax.ShapeDtypeStruct((N, 32), f32),
    )(t0, t1, b)


_GDN = lax.GatherDimensionNumbers(
    offset_dims=(), collapsed_slice_dims=(0,), start_index_map=(0,))


def _bcast_lane(ev, lanei):
    sel = lax.broadcast(lanei, (16,)).astype(i32).reshape(16, 1)
    return lax.gather(ev, sel, dimension_numbers=_GDN, slice_sizes=(1,),
                      mode=lax.GatherScatterMode.PROMISE_IN_BOUNDS)


def _make_sc_scores(EP, HS):
    EPT = EP // 16
    SUP = EPT // 1024
    mesh = plsc.VectorSubcoreMesh(core_axis_name="c", subcore_axis_name="s")

    @functools.partial(
        pl.kernel,
        out_type=jax.ShapeDtypeStruct((2, EP // 64, 64 * HS), f32),
        mesh=mesh,
        compiler_params=_SC_PARAMS,
        scratch_types=[
            pltpu.VMEM((RT * HS,), f32),
            pltpu.VMEM((RT * HS,), f32),
            pltpu.VMEM((16, 64), i32), pltpu.VMEM((16, 64), i32),
            pltpu.VMEM((16, 64), i32), pltpu.VMEM((16, 64), i32),
            pltpu.VMEM((16, 64 * HS), f32), pltpu.VMEM((16, 64 * HS), f32),
            pltpu.SemaphoreType.DMA, pltpu.SemaphoreType.DMA,
            pltpu.SemaphoreType.DMA, pltpu.SemaphoreType.DMA,
        ],
    )
    def scores(as_hbm, ad_hbm, src_hbm, dst_hbm, ex_hbm,
               as_v, ad_v, srcA, srcB, dstA, dstB, exA, exB,
               lsemA, lsemB, ssemA, ssemB):
        c = lax.axis_index("c")
        s = lax.axis_index("s")
        pltpu.sync_copy(as_hbm.at[c], as_v)
        pltpu.sync_copy(ad_hbm.at[c], ad_v)

        srcs = [srcA, srcB]
        dsts = [dstA, dstB]
        exs = [exA, exB]
        lsems = [lsemA, lsemB]
        ssems = [ssemA, ssemB]

        def issue_loads(g):
            b = g % 2
            base64 = s * (EPT // 64) + g * 16
            d1 = pltpu.async_copy(src_hbm.at[c, pl.ds(base64, 16)],
                                  srcs[b], lsems[b])
            d2 = pltpu.async_copy(dst_hbm.at[c, pl.ds(base64, 16)],
                                  dsts[b], lsems[b])
            return (d1, d2)

        lds = {0: issue_loads(0)}
        sts = {}
        for g in range(SUP):
            b = g % 2
            if g + 1 < SUP:
                lds[g + 1] = issue_loads(g + 1)
            lds[g][0].wait()
            lds[g][1].wait()
            if g >= 2:
                sts[g - 2].wait()
            src_v, dst_v, ex_v = srcs[b], dsts[b], exs[b]

            def row(r, _):
                for t in range(4):
                    sv = src_v[r, pl.ds(t * 16, 16)]
                    dv = dst_v[r, pl.ds(t * 16, 16)]
                    for h in range(HS):
                        a = (plsc.load_gather(as_v, [sv * HS + h]) +
                             plsc.load_gather(ad_v, [dv * HS + h]))
                        a = jnp.where(a > 0, a, 0.2 * a)
                        ex_v[r, pl.ds(h * 64 + t * 16, 16)] = jnp.exp(a)
                return 0

            lax.fori_loop(0, 16, row, 0)
            base64 = s * (EPT // 64) + g * 16
            sts[g] = pltpu.async_copy(
                ex_v, ex_hbm.at[c, pl.ds(base64, 16)], ssems[b])
        sts[SUP - 2].wait()
        sts[SUP - 1].wait()

    return scores


def _make_sc_msg(EP, HS, F):
    W = F + 16
    CH = F // HS
    EPT = EP // 16
    SUP = EPT // 1024
    RPT = RO // 16
    mesh = plsc.VectorSubcoreMesh(core_axis_name="c", subcore_axis_name="s")

    @functools.partial(
        pl.kernel,
        out_type=jax.ShapeDtypeStruct((2, RO, W), f32),
        mesh=mesh,
        compiler_params=_SC_PARAMS,
        scratch_types=[
            pltpu.VMEM((16, 64), i32),
            pltpu.VMEM((16, 64), i32),
            pltpu.VMEM((16, 64), i32),
            pltpu.VMEM((16, 64 * HS), f32),
            pltpu.VMEM((64, F), f32), pltpu.VMEM((64, F), f32),
            pltpu.VMEM((64, W), f32), pltpu.VMEM((64, W), f32),
            pltpu.VMEM_SHARED((RO, W), f32),
            pltpu.SemaphoreType.DMA,
            pltpu.SemaphoreType.DMA, pltpu.SemaphoreType.DMA,
            pltpu.SemaphoreType.DMA, pltpu.SemaphoreType.DMA,
        ],
    )
    def msg(hs_hbm, ex_hbm, src_hbm, dst_hbm, out_hbm,
            src_v, dst_v, srcg_v, ex_v, rowsA, rowsB, orowA, orowB,
            out_sh, lsem, gsemA, gsemB, ssemA, ssemB):
        c = lax.axis_index("c")
        s = lax.axis_index("s")

        zero16 = jnp.zeros((16,), f32)
        rows = [rowsA, rowsB]
        orows = [orowA, orowB]
        gsems = [gsemA, gsemB]
        ssems = [ssemA, ssemB]

        def zero_row(j, _):
            for t in range(W // 16):
                orowA[j, pl.ds(t * 16, 16)] = zero16
            return 0

        lax.fori_loop(0, 64, zero_row, 0)
        r0 = s * RPT
        for t in range(9):
            pltpu.sync_copy(orowA, out_sh.at[pl.ds(r0 + t * 64, 64)])
        pltpu.sync_copy(orowA.at[pl.ds(0, RPT - 576)],
                        out_sh.at[pl.ds(r0 + 576, RPT - 576)])
        plsc.subcore_barrier()

        offv = lax.broadcast(c * RT, (16,)).astype(i32)
        lane = lax.iota(i32, 16)

        def sup(g, _):
            base64 = s * (EPT // 64) + g * 16
            d1 = pltpu.async_copy(src_hbm.at[c, pl.ds(base64, 16)],
                                  src_v, lsem)
            d2 = pltpu.async_copy(dst_hbm.at[c, pl.ds(base64, 16)],
                                  dst_v, lsem)
            d3 = pltpu.async_copy(ex_hbm.at[c, pl.ds(base64, 16)],
                                  ex_v, lsem)
            d1.wait(); d2.wait(); d3.wait()

            def adj(r, _):
                for t in range(4):
                    srcg_v[r, pl.ds(t * 16, 16)] = (
                        src_v[r, pl.ds(t * 16, 16)] + offv)
                return 0

            lax.fori_loop(0, 16, adj, 0)

            gd = {0: pltpu.async_copy(hs_hbm.at[srcg_v.at[0]],
                                      rows[0], gsems[0])}
            for j in range(16):
                b = j % 2
                if j + 1 < 16:
                    gd[j + 1] = pltpu.async_copy(
                        hs_hbm.at[srcg_v.at[j + 1]],
                        rows[(j + 1) % 2], gsems[(j + 1) % 2])
                gd[j].wait()
                rb = rows[b]
                ob = orows[b]

                def quad(q, _):
                    g16 = (q // 4) * 16
                    evs = [ex_v[j, pl.ds(h * 64 + g16, 16)]
                           for h in range(HS)]
                    for k in range(4):
                        lanei = (q % 4) * 4 + k
                        bexs = [_bcast_lane(ev, lanei) for ev in evs]
                        e = q * 4 + k
                        for t in range(F // 16):
                            h = (t * 16) // CH
                            ob[e, pl.ds(t * 16, 16)] = (
                                rb[e, pl.ds(t * 16, 16)] * bexs[h])
                        tail = zero16
                        for h in range(HS):
                            tail = jnp.where(lane == h, bexs[h], tail)
                        ob[e, pl.ds(F, 16)] = tail
                    return 0

                lax.fori_loop(0, 16, quad, 0)
                pltpu.sync_copy(ob, out_sh.at[dst_v.at[j]], add=True)
            return 0

        lax.fori_loop(0, SUP, sup, 0)
        plsc.subcore_barrier()
        for t in range(9):
            pltpu.sync_copy(out_sh.at[pl.ds(r0 + t * 64, 64)], orowA)
            pltpu.sync_copy(orowA, out_hbm.at[c, pl.ds(r0 + t * 64, 64)])
        pltpu.sync_copy(out_sh.at[pl.ds(r0 + 576, RPT - 576)],
                        orowA.at[pl.ds(0, RPT - 576)])
        pltpu.sync_copy(orowA.at[pl.ds(0, RPT - 576)],
                        out_hbm.at[c, pl.ds(r0 + 576, RPT - 576)])

    return msg


_sc_scores_l0 = _make_sc_scores(EP=327680, HS=2)
_sc_scores_l1 = _make_sc_scores(EP=163840, HS=1)
_sc_msg_l0 = _make_sc_msg(EP=327680, HS=2, F=128)
_sc_msg_l1 = _make_sc_msg(EP=163840, HS=1, F=32)


def _sc_conv_l0(hs_t, as_t, ad_t, src, dst):
    ex = _sc_scores_l0(as_t, ad_t, src, dst)
    return _sc_msg_l0(hs_t, ex, src, dst)


def _sc_conv_l1(hs_t, as_t, ad_t, src, dst):
    ex = _sc_scores_l1(as_t, ad_t, src, dst)
    return _sc_msg_l1(hs_t, ex, src, dst)


def _pad_rows(x, val=0.0):
    return jnp.concatenate(
        [x, jnp.full((RT - N,) + x.shape[1:], val, x.dtype)], axis=0)


def _prep_edges_l0(ei):
    src = ei[0].astype(i32)
    dst = ei[1].astype(i32)
    pad = jnp.full((327680 - E,), N, i32)
    srcp = jnp.concatenate([src, pad]).reshape(327680 // 64, 64)
    dstp = jnp.concatenate([dst, pad]).reshape(327680 // 64, 64)
    return (jnp.broadcast_to(srcp[None], (2, 327680 // 64, 64)),
            jnp.broadcast_to(dstp[None], (2, 327680 // 64, 64)))


def _prep_edges_l1(ei):
    src = ei[0].astype(i32)
    dst = ei[1].astype(i32)
    half = E // 2
    pad = jnp.full((163840 - half,), N, i32)
    srcp = jnp.stack([jnp.concatenate([src[:half], pad]),
                      jnp.concatenate([src[half:], pad])])
    dstp = jnp.stack([jnp.concatenate([dst[:half], pad]),
                      jnp.concatenate([dst[half:], pad])])
    return (srcp.reshape(2, 163840 // 64, 64),
            dstp.reshape(2, 163840 // 64, 64))


def _score_tables_l0(a):
    ap = _pad_rows(a, NEG)
    return ap.reshape(RT, 2, 2).transpose(1, 0, 2).reshape(2, RT * 2)


def _score_tables_l1(a):
    ap = _pad_rows(a[:, None], NEG).reshape(RT)
    return jnp.broadcast_to(ap[None], (2, RT))


def _hs_table_l0(hs):
    hp = _pad_rows(hs)
    return jnp.concatenate([hp[:, :128], hp[:, 128:]], axis=0)


def _hs_table_l1(hs):
    hp = _pad_rows(hs)
    return jnp.concatenate([hp, hp], axis=0)


def _fold_l0(Wsrc, avec):
    return jnp.einsum('khc,hc->kh', Wsrc.reshape(64, 4, 64), avec)


def kernel(x_user, x_item, edge_index_u2i, edge_index_i2u,
           W_in_user, b_in_user, W_in_item, b_in_item,
           l0_u2i_Wsrc, l0_u2i_Wdst, l0_u2i_asrc, l0_u2i_adst, l0_u2i_b,
           l0_i2u_Wsrc, l0_i2u_Wdst, l0_i2u_asrc, l0_i2u_adst, l0_i2u_b,
           l1_u2i_Wsrc, l1_u2i_Wdst, l1_u2i_asrc, l1_u2i_adst, l1_u2i_b,
           l1_i2u_Wsrc, l1_i2u_Wdst, l1_i2u_asrc, l1_i2u_adst, l1_i2u_b):
    ws0_u2i = _fold_l0(l0_u2i_Wsrc, l0_u2i_asrc)
    wd0_u2i = _fold_l0(l0_u2i_Wdst, l0_u2i_adst)
    ws0_i2u = _fold_l0(l0_i2u_Wsrc, l0_i2u_asrc)
    wd0_i2u = _fold_l0(l0_i2u_Wdst, l0_i2u_adst)
    WbigU = jnp.concatenate([l0_u2i_Wsrc, ws0_u2i, wd0_i2u], axis=1)
    WbigI = jnp.concatenate([l0_i2u_Wsrc, ws0_i2u, wd0_u2i], axis=1)

    bigU = _tc_in_proj(x_user, W_in_user, b_in_user[None], WbigU)
    bigI = _tc_in_proj(x_item, W_in_item, b_in_item[None], WbigI)

    su0, du0 = _prep_edges_l0(edge_index_u2i)
    si0, di0 = _prep_edges_l0(edge_index_i2u)

    outU = _sc_conv_l0(_hs_table_l0(bigU[:, :256]),
                       _score_tables_l0(bigU[:, 256:260]),
                       _score_tables_l0(bigI[:, 260:264]),
                       su0, du0)
    outI = _sc_conv_l0(_hs_table_l0(bigI[:, :256]),
                       _score_tables_l0(bigI[:, 256:260]),
                       _score_tables_l0(bigU[:, 260:264]),
                       si0, di0)

    num_i = jnp.concatenate([outU[0, :N, :128], outU[1, :N, :128]], axis=1)
    den_i = jnp.concatenate([outU[0, :N, 128:130], outU[1, :N, 128:130]], axis=1)
    num_u = jnp.concatenate([outI[0, :N, :128], outI[1, :N, :128]], axis=1)
    den_u = jnp.concatenate([outI[0, :N, 128:130], outI[1, :N, 128:130]], axis=1)

    ws1_u2i = l1_u2i_Wsrc @ l1_u2i_asrc[0]
    wd1_u2i = l1_u2i_Wdst @ l1_u2i_adst[0]
    ws1_i2u = l1_i2u_Wsrc @ l1_i2u_asrc[0]
    wd1_i2u = l1_i2u_Wdst @ l1_i2u_adst[0]
    W1bigU = jnp.concatenate(
        [l1_u2i_Wsrc, ws1_u2i[:, None], wd1_i2u[:, None]], axis=1)
    W1bigI = jnp.concatenate(
        [l1_i2u_Wsrc, ws1_i2u[:, None], wd1_u2i[:, None]], axis=1)

    big1U = _tc_mid(num_u, den_u, l0_i2u_b[None], W1bigU)
    big1I = _tc_mid(num_i, den_i, l0_u2i_b[None], W1bigI)

    su1, du1 = _prep_edges_l1(edge_index_u2i)
    si1, di1 = _prep_edges_l1(edge_index_i2u)

    out1U = _sc_conv_l1(_hs_table_l1(big1U[:, :32]),
                        _score_tables_l1(big1U[:, 32]),
                        _score_tables_l1(big1I[:, 33]),
                        su1, du1)
    out1I = _sc_conv_l1(_hs_table_l1(big1I[:, :32]),
                        _score_tables_l1(big1I[:, 32]),
                        _score_tables_l1(big1U[:, 33]),
                        si1, di1)

    hi2 = _tc_final(out1U[0, :N, :33], out1U[1, :N, :33], l1_u2i_b[None])
    hu2 = _tc_final(out1I[0, :N, :33], out1I[1, :N, :33], l1_i2u_b[None])
    return hu2, hi2


# speedup vs baseline: 36.0680x; 1.0739x over previous
import functools

import jax
import jax.numpy as jnp
from jax import lax
from jax.experimental import pallas as pl
from jax.experimental.pallas import tpu as pltpu
from jax.experimental.pallas import tpu_sc as plsc

N = 10000
E = 320000
RT = 10016
RO = 10016
NEG = -1.0e30

f32 = jnp.float32
i32 = jnp.int32

_SC_PARAMS = pltpu.CompilerParams(
    needs_layout_passes=False, use_tc_tiling_on_sc=False)


def _tc_in_proj(x, W1, b1, Wbig):
    B = 400
    K2 = Wbig.shape[1]

    def body(x_r, w1_r, b1_r, wb_r, o_r):
        h = jnp.dot(x_r[...], w1_r[...], preferred_element_type=f32) + b1_r[...]
        h = jnp.where(h > 0, h, jnp.exp(jnp.minimum(h, 0.0)) - 1.0)
        o_r[...] = jnp.dot(h, wb_r[...], preferred_element_type=f32)

    return pl.pallas_call(
        body,
        grid=(N // B,),
        in_specs=[
            pl.BlockSpec((B, x.shape[1]), lambda i: (i, 0)),
            pl.BlockSpec(W1.shape, lambda i: (0, 0)),
            pl.BlockSpec(b1.shape, lambda i: (0, 0)),
            pl.BlockSpec(Wbig.shape, lambda i: (0, 0)),
        ],
        out_specs=pl.BlockSpec((B, K2), lambda i: (i, 0)),
        out_shape=jax.ShapeDtypeStruct((N, K2), f32),
    )(x, W1, b1, Wbig)


def _tc_mid(num, den, b, Wbig):
    B = 400
    K2 = Wbig.shape[1]

    def body(n_r, d_r, b_r, wb_r, o_r):
        d = d_r[...]
        drep = jnp.concatenate(
            [jnp.broadcast_to(d[:, h:h + 1], (B, 64)) for h in range(4)], axis=1)
        t = n_r[...] / (drep + 1e-16) + b_r[...]
        t = jnp.where(t > 0, t, jnp.exp(jnp.minimum(t, 0.0)) - 1.0)
        o_r[...] = jnp.dot(t, wb_r[...], preferred_element_type=f32)

    return pl.pallas_call(
        body,
        grid=(N // B,),
        in_specs=[
            pl.BlockSpec((B, 256), lambda i: (i, 0)),
            pl.BlockSpec((B, 4), lambda i: (i, 0)),
            pl.BlockSpec(b.shape, lambda i: (0, 0)),
            pl.BlockSpec(Wbig.shape, lambda i: (0, 0)),
        ],
        out_specs=pl.BlockSpec((B, K2), lambda i: (i, 0)),
        out_shape=jax.ShapeDtypeStruct((N, K2), f32),
    )(num, den, b, Wbig)


def _tc_final(t0, t1, b):
    B = 1000

    def body(a_r, c_r, b_r, o_r):
        s = a_r[...] + c_r[...]
        o_r[...] = s[:, :32] / (s[:, 32:33] + 1e-16) + b_r[...]

    return pl.pallas_call(
        body,
        grid=(N // B,),
        in_specs=[
            pl.BlockSpec((B, 33), lambda i: (i, 0)),
            pl.BlockSpec((B, 33), lambda i: (i, 0)),
            pl.BlockSpec(b.shape, lambda i: (0, 0)),
        ],
        out_specs=pl.BlockSpec((B, 32), lambda i: (i, 0)),
        out_shape=jax.ShapeDtypeStruct((N, 32), f32),
    )(t0, t1, b)


_GDN = lax.GatherDimensionNumbers(
    offset_dims=(), collapsed_slice_dims=(0,), start_index_map=(0,))


def _bcast_lane(ev, lanei):
    sel = lax.broadcast(lanei, (16,)).astype(i32).reshape(16, 1)
    return lax.gather(ev, sel, dimension_numbers=_GDN, slice_sizes=(1,),
                      mode=lax.GatherScatterMode.PROMISE_IN_BOUNDS)


def _make_sc_scores(EP, HS):
    EPT = EP // 16
    SUP = EPT // 1024
    mesh = plsc.VectorSubcoreMesh(core_axis_name="c", subcore_axis_name="s")

    @functools.partial(
        pl.kernel,
        out_type=jax.ShapeDtypeStruct((2, EP // 64, 64 * HS), f32),
        mesh=mesh,
        compiler_params=_SC_PARAMS,
        scratch_types=[
            pltpu.VMEM((RT * HS,), f32),
            pltpu.VMEM((RT * HS,), f32),
            pltpu.VMEM((16, 64), i32), pltpu.VMEM((16, 64), i32),
            pltpu.VMEM((16, 64), i32), pltpu.VMEM((16, 64), i32),
            pltpu.VMEM((16, 64 * HS), f32), pltpu.VMEM((16, 64 * HS), f32),
            pltpu.SemaphoreType.DMA, pltpu.SemaphoreType.DMA,
            pltpu.SemaphoreType.DMA, pltpu.SemaphoreType.DMA,
        ],
    )
    def scores(as_hbm, ad_hbm, src_hbm, dst_hbm, ex_hbm,
               as_v, ad_v, srcA, srcB, dstA, dstB, exA, exB,
               lsemA, lsemB, ssemA, ssemB):
        c = lax.axis_index("c")
        s = lax.axis_index("s")
        pltpu.sync_copy(as_hbm.at[c], as_v)
        pltpu.sync_copy(ad_hbm.at[c], ad_v)

        srcs = [srcA, srcB]
        dsts = [dstA, dstB]
        exs = [exA, exB]
        lsems = [lsemA, lsemB]
        ssems = [ssemA, ssemB]

        def issue_loads(g):
            b = g % 2
            base64 = s * (EPT // 64) + g * 16
            d1 = pltpu.async_copy(src_hbm.at[c, pl.ds(base64, 16)],
                                  srcs[b], lsems[b])
            d2 = pltpu.async_copy(dst_hbm.at[c, pl.ds(base64, 16)],
                                  dsts[b], lsems[b])
            return (d1, d2)

        lds = {0: issue_loads(0)}
        sts = {}
        for g in range(SUP):
            b = g % 2
            if g + 1 < SUP:
                lds[g + 1] = issue_loads(g + 1)
            lds[g][0].wait()
            lds[g][1].wait()
            if g >= 2:
                sts[g - 2].wait()
            src_v, dst_v, ex_v = srcs[b], dsts[b], exs[b]

            def row(r, _):
                for t in range(4):
                    sv = src_v[r, pl.ds(t * 16, 16)]
                    dv = dst_v[r, pl.ds(t * 16, 16)]
                    for h in range(HS):
                        a = (plsc.load_gather(as_v, [sv * HS + h]) +
                             plsc.load_gather(ad_v, [dv * HS + h]))
                        a = jnp.where(a > 0, a, 0.2 * a)
                        ex_v[r, pl.ds(h * 64 + t * 16, 16)] = jnp.exp(a)
                return 0

            lax.fori_loop(0, 16, row, 0)
            base64 = s * (EPT // 64) + g * 16
            sts[g] = pltpu.async_copy(
                ex_v, ex_hbm.at[c, pl.ds(base64, 16)], ssems[b])
        sts[SUP - 2].wait()
        sts[SUP - 1].wait()

    return scores


def _make_sc_msg(EP, HS, F):
    W = F + 16
    CH = F // HS
    EPT = EP // 16
    SUP = EPT // 1024
    RPT = RO // 16
    mesh = plsc.VectorSubcoreMesh(core_axis_name="c", subcore_axis_name="s")

    @functools.partial(
        pl.kernel,
        out_type=jax.ShapeDtypeStruct((2, RO, W), f32),
        mesh=mesh,
        compiler_params=_SC_PARAMS,
        scratch_types=[
            pltpu.VMEM((16, 64), i32),
            pltpu.VMEM((16, 64), i32),
            pltpu.VMEM((16, 64), i32),
            pltpu.VMEM((16, 64 * HS), f32),
            pltpu.VMEM((64, F), f32), pltpu.VMEM((64, F), f32),
            pltpu.VMEM((64, W), f32), pltpu.VMEM((64, W), f32),
            pltpu.VMEM_SHARED((RO, W), f32),
            pltpu.SemaphoreType.DMA,
            pltpu.SemaphoreType.DMA, pltpu.SemaphoreType.DMA,
            pltpu.SemaphoreType.DMA, pltpu.SemaphoreType.DMA,
        ],
    )
    def msg(hs_hbm, ex_hbm, src_hbm, dst_hbm, out_hbm,
            src_v, dst_v, srcg_v, ex_v, rowsA, rowsB, orowA, orowB,
            out_sh, lsem, gsemA, gsemB, ssemA, ssemB):
        c = lax.axis_index("c")
        s = lax.axis_index("s")

        zero16 = jnp.zeros((16,), f32)
        rows = [rowsA, rowsB]
        orows = [orowA, orowB]
        gsems = [gsemA, gsemB]
        ssems = [ssemA, ssemB]

        def zero_row(j, _):
            for t in range(W // 16):
                orowA[j, pl.ds(t * 16, 16)] = zero16
            return 0

        lax.fori_loop(0, 64, zero_row, 0)
        r0 = s * RPT
        for t in range(9):
            pltpu.sync_copy(orowA, out_sh.at[pl.ds(r0 + t * 64, 64)])
        pltpu.sync_copy(orowA.at[pl.ds(0, RPT - 576)],
                        out_sh.at[pl.ds(r0 + 576, RPT - 576)])
        plsc.subcore_barrier()

        offv = lax.broadcast(c * RT, (16,)).astype(i32)
        lane = lax.iota(i32, 16)

        def sup(g, _):
            base64 = s * (EPT // 64) + g * 16
            d1 = pltpu.async_copy(src_hbm.at[c, pl.ds(base64, 16)],
                                  src_v, lsem)
            d2 = pltpu.async_copy(dst_hbm.at[c, pl.ds(base64, 16)],
                                  dst_v, lsem)
            d3 = pltpu.async_copy(ex_hbm.at[c, pl.ds(base64, 16)],
                                  ex_v, lsem)
            d1.wait(); d2.wait(); d3.wait()

            def adj(r, _):
                for t in range(4):
                    srcg_v[r, pl.ds(t * 16, 16)] = (
                        src_v[r, pl.ds(t * 16, 16)] + offv)
                return 0

            lax.fori_loop(0, 16, adj, 0)

            gd = {0: pltpu.async_copy(hs_hbm.at[srcg_v.at[0]],
                                      rows[0], gsems[0])}
            sd = {}
            for j in range(16):
                b = j % 2
                if j + 1 < 16:
                    gd[j + 1] = pltpu.async_copy(
                        hs_hbm.at[srcg_v.at[j + 1]],
                        rows[(j + 1) % 2], gsems[(j + 1) % 2])
                gd[j].wait()
                rb = rows[b]
                ob = orows[b]

                def quad(q, _):
                    g16 = (q // 4) * 16
                    evs = [ex_v[j, pl.ds(h * 64 + g16, 16)]
                           for h in range(HS)]
                    for k in range(4):
                        lanei = (q % 4) * 4 + k
                        bexs = [_bcast_lane(ev, lanei) for ev in evs]
                        e = q * 4 + k
                        for t in range(F // 16):
                            h = (t * 16) // CH
                            ob[e, pl.ds(t * 16, 16)] = (
                                rb[e, pl.ds(t * 16, 16)] * bexs[h])
                        tail = zero16
                        for h in range(HS):
                            tail = jnp.where(lane == h, bexs[h], tail)
                        ob[e, pl.ds(F, 16)] = tail
                    return 0

                lax.fori_loop(0, 16, quad, 0)
                if j >= 1:
                    sd[j - 1].wait()
                sd[j] = pltpu.async_copy(
                    ob, out_sh.at[dst_v.at[j]], ssems[b], add=True)
            sd[15].wait()
            return 0

        lax.fori_loop(0, SUP, sup, 0)
        plsc.subcore_barrier()
        for t in range(9):
            pltpu.sync_copy(out_sh.at[pl.ds(r0 + t * 64, 64)], orowA)
            pltpu.sync_copy(orowA, out_hbm.at[c, pl.ds(r0 + t * 64, 64)])
        pltpu.sync_copy(out_sh.at[pl.ds(r0 + 576, RPT - 576)],
                        orowA.at[pl.ds(0, RPT - 576)])
        pltpu.sync_copy(orowA.at[pl.ds(0, RPT - 576)],
                        out_hbm.at[c, pl.ds(r0 + 576, RPT - 576)])

    return msg


_sc_scores_l0 = _make_sc_scores(EP=327680, HS=2)
_sc_scores_l1 = _make_sc_scores(EP=163840, HS=1)
_sc_msg_l0 = _make_sc_msg(EP=327680, HS=2, F=128)
_sc_msg_l1 = _make_sc_msg(EP=163840, HS=1, F=32)


def _sc_conv_l0(hs_t, as_t, ad_t, src, dst):
    ex = _sc_scores_l0(as_t, ad_t, src, dst)
    return _sc_msg_l0(hs_t, ex, src, dst)


def _sc_conv_l1(hs_t, as_t, ad_t, src, dst):
    ex = _sc_scores_l1(as_t, ad_t, src, dst)
    return _sc_msg_l1(hs_t, ex, src, dst)


def _pad_rows(x, val=0.0):
    return jnp.concatenate(
        [x, jnp.full((RT - N,) + x.shape[1:], val, x.dtype)], axis=0)


def _prep_edges_l0(ei):
    src = ei[0].astype(i32)
    dst = ei[1].astype(i32)
    pad = jnp.full((327680 - E,), N, i32)
    srcp = jnp.concatenate([src, pad]).reshape(327680 // 64, 64)
    dstp = jnp.concatenate([dst, pad]).reshape(327680 // 64, 64)
    return (jnp.broadcast_to(srcp[None], (2, 327680 // 64, 64)),
            jnp.broadcast_to(dstp[None], (2, 327680 // 64, 64)))


def _prep_edges_l1(ei):
    src = ei[0].astype(i32)
    dst = ei[1].astype(i32)
    half = E // 2
    pad = jnp.full((163840 - half,), N, i32)
    srcp = jnp.stack([jnp.concatenate([src[:half], pad]),
                      jnp.concatenate([src[half:], pad])])
    dstp = jnp.stack([jnp.concatenate([dst[:half], pad]),
                      jnp.concatenate([dst[half:], pad])])
    return (srcp.reshape(2, 163840 // 64, 64),
            dstp.reshape(2, 163840 // 64, 64))


def _score_tables_l0(a):
    ap = _pad_rows(a, NEG)
    return ap.reshape(RT, 2, 2).transpose(1, 0, 2).reshape(2, RT * 2)


def _score_tables_l1(a):
    ap = _pad_rows(a[:, None], NEG).reshape(RT)
    return jnp.broadcast_to(ap[None], (2, RT))


def _hs_table_l0(hs):
    hp = _pad_rows(hs)
    return jnp.concatenate([hp[:, :128], hp[:, 128:]], axis=0)


def _hs_table_l1(hs):
    hp = _pad_rows(hs)
    return jnp.concatenate([hp, hp], axis=0)


def _fold_l0(Wsrc, avec):
    return jnp.einsum('khc,hc->kh', Wsrc.reshape(64, 4, 64), avec)


def kernel(x_user, x_item, edge_index_u2i, edge_index_i2u,
           W_in_user, b_in_user, W_in_item, b_in_item,
           l0_u2i_Wsrc, l0_u2i_Wdst, l0_u2i_asrc, l0_u2i_adst, l0_u2i_b,
           l0_i2u_Wsrc, l0_i2u_Wdst, l0_i2u_asrc, l0_i2u_adst, l0_i2u_b,
           l1_u2i_Wsrc, l1_u2i_Wdst, l1_u2i_asrc, l1_u2i_adst, l1_u2i_b,
           l1_i2u_Wsrc, l1_i2u_Wdst, l1_i2u_asrc, l1_i2u_adst, l1_i2u_b):
    ws0_u2i = _fold_l0(l0_u2i_Wsrc, l0_u2i_asrc)
    wd0_u2i = _fold_l0(l0_u2i_Wdst, l0_u2i_adst)
    ws0_i2u = _fold_l0(l0_i2u_Wsrc, l0_i2u_asrc)
    wd0_i2u = _fold_l0(l0_i2u_Wdst, l0_i2u_adst)
    WbigU = jnp.concatenate([l0_u2i_Wsrc, ws0_u2i, wd0_i2u], axis=1)
    WbigI = jnp.concatenate([l0_i2u_Wsrc, ws0_i2u, wd0_u2i], axis=1)

    bigU = _tc_in_proj(x_user, W_in_user, b_in_user[None], WbigU)
    bigI = _tc_in_proj(x_item, W_in_item, b_in_item[None], WbigI)

    su0, du0 = _prep_edges_l0(edge_index_u2i)
    si0, di0 = _prep_edges_l0(edge_index_i2u)

    outU = _sc_conv_l0(_hs_table_l0(bigU[:, :256]),
                       _score_tables_l0(bigU[:, 256:260]),
                       _score_tables_l0(bigI[:, 260:264]),
                       su0, du0)
    outI = _sc_conv_l0(_hs_table_l0(bigI[:, :256]),
                       _score_tables_l0(bigI[:, 256:260]),
                       _score_tables_l0(bigU[:, 260:264]),
                       si0, di0)

    num_i = jnp.concatenate([outU[0, :N, :128], outU[1, :N, :128]], axis=1)
    den_i = jnp.concatenate([outU[0, :N, 128:130], outU[1, :N, 128:130]], axis=1)
    num_u = jnp.concatenate([outI[0, :N, :128], outI[1, :N, :128]], axis=1)
    den_u = jnp.concatenate([outI[0, :N, 128:130], outI[1, :N, 128:130]], axis=1)

    ws1_u2i = l1_u2i_Wsrc @ l1_u2i_asrc[0]
    wd1_u2i = l1_u2i_Wdst @ l1_u2i_adst[0]
    ws1_i2u = l1_i2u_Wsrc @ l1_i2u_asrc[0]
    wd1_i2u = l1_i2u_Wdst @ l1_i2u_adst[0]
    W1bigU = jnp.concatenate(
        [l1_u2i_Wsrc, ws1_u2i[:, None], wd1_i2u[:, None]], axis=1)
    W1bigI = jnp.concatenate(
        [l1_i2u_Wsrc, ws1_i2u[:, None], wd1_u2i[:, None]], axis=1)

    big1U = _tc_mid(num_u, den_u, l0_i2u_b[None], W1bigU)
    big1I = _tc_mid(num_i, den_i, l0_u2i_b[None], W1bigI)

    su1, du1 = _prep_edges_l1(edge_index_u2i)
    si1, di1 = _prep_edges_l1(edge_index_i2u)

    out1U = _sc_conv_l1(_hs_table_l1(big1U[:, :32]),
                        _score_tables_l1(big1U[:, 32]),
                        _score_tables_l1(big1I[:, 33]),
                        su1, du1)
    out1I = _sc_conv_l1(_hs_table_l1(big1I[:, :32]),
                        _score_tables_l1(big1I[:, 32]),
                        _score_tables_l1(big1U[:, 33]),
                        si1, di1)

    hi2 = _tc_final(out1U[0, :N, :33], out1U[1, :N, :33], l1_u2i_b[None])
    hu2 = _tc_final(out1I[0, :N, :33], out1I[1, :N, :33], l1_i2u_b[None])
    return hu2, hi2


# speedup vs baseline: 36.2181x; 1.0042x over previous
import functools

import jax
import jax.numpy as jnp
from jax import lax
from jax.experimental import pallas as pl
from jax.experimental.pallas import tpu as pltpu
from jax.experimental.pallas import tpu_sc as plsc

N = 10000
E = 320000
RT = 10016
RO = 10016
NEG = -1.0e30

f32 = jnp.float32
i32 = jnp.int32

_SC_PARAMS = pltpu.CompilerParams(
    needs_layout_passes=False, use_tc_tiling_on_sc=False)


def _tc_in_proj(x, W1, b1, Wbig):
    B = 400
    K2 = Wbig.shape[1]

    def body(x_r, w1_r, b1_r, wb_r, o_r):
        h = jnp.dot(x_r[...], w1_r[...], preferred_element_type=f32) + b1_r[...]
        h = jnp.where(h > 0, h, jnp.exp(jnp.minimum(h, 0.0)) - 1.0)
        o_r[...] = jnp.dot(h, wb_r[...], preferred_element_type=f32)

    return pl.pallas_call(
        body,
        grid=(N // B,),
        in_specs=[
            pl.BlockSpec((B, x.shape[1]), lambda i: (i, 0)),
            pl.BlockSpec(W1.shape, lambda i: (0, 0)),
            pl.BlockSpec(b1.shape, lambda i: (0, 0)),
            pl.BlockSpec(Wbig.shape, lambda i: (0, 0)),
        ],
        out_specs=pl.BlockSpec((B, K2), lambda i: (i, 0)),
        out_shape=jax.ShapeDtypeStruct((N, K2), f32),
    )(x, W1, b1, Wbig)


def _tc_mid(num, den, b, Wbig):
    B = 400
    K2 = Wbig.shape[1]

    def body(n_r, d_r, b_r, wb_r, o_r):
        d = d_r[...]
        drep = jnp.concatenate(
            [jnp.broadcast_to(d[:, h:h + 1], (B, 64)) for h in range(4)], axis=1)
        t = n_r[...] / (drep + 1e-16) + b_r[...]
        t = jnp.where(t > 0, t, jnp.exp(jnp.minimum(t, 0.0)) - 1.0)
        o_r[...] = jnp.dot(t, wb_r[...], preferred_element_type=f32)

    return pl.pallas_call(
        body,
        grid=(N // B,),
        in_specs=[
            pl.BlockSpec((B, 256), lambda i: (i, 0)),
            pl.BlockSpec((B, 4), lambda i: (i, 0)),
            pl.BlockSpec(b.shape, lambda i: (0, 0)),
            pl.BlockSpec(Wbig.shape, lambda i: (0, 0)),
        ],
        out_specs=pl.BlockSpec((B, K2), lambda i: (i, 0)),
        out_shape=jax.ShapeDtypeStruct((N, K2), f32),
    )(num, den, b, Wbig)


def _tc_final(t0, t1, b):
    B = 1000

    def body(a_r, c_r, b_r, o_r):
        s = a_r[...] + c_r[...]
        o_r[...] = s[:, :32] / (s[:, 32:33] + 1e-16) + b_r[...]

    return pl.pallas_call(
        body,
        grid=(N // B,),
        in_specs=[
            pl.BlockSpec((B, 33), lambda i: (i, 0)),
            pl.BlockSpec((B, 33), lambda i: (i, 0)),
            pl.BlockSpec(b.shape, lambda i: (0, 0)),
        ],
        out_specs=pl.BlockSpec((B, 32), lambda i: (i, 0)),
        out_shape=jax.ShapeDtypeStruct((N, 32), f32),
    )(t0, t1, b)


_GDN = lax.GatherDimensionNumbers(
    offset_dims=(), collapsed_slice_dims=(0,), start_index_map=(0,))


def _bcast_lane(ev, lanei):
    sel = lax.broadcast(lanei, (16,)).astype(i32).reshape(16, 1)
    return lax.gather(ev, sel, dimension_numbers=_GDN, slice_sizes=(1,),
                      mode=lax.GatherScatterMode.PROMISE_IN_BOUNDS)


def _make_sc_scores(EP, HS):
    EPT = EP // 16
    SUP = EPT // 1024
    mesh = plsc.VectorSubcoreMesh(core_axis_name="c", subcore_axis_name="s")

    @functools.partial(
        pl.kernel,
        out_type=jax.ShapeDtypeStruct((2, EP // 64, 64 * HS), f32),
        mesh=mesh,
        compiler_params=_SC_PARAMS,
        scratch_types=[
            pltpu.VMEM((RT * HS,), f32),
            pltpu.VMEM((RT * HS,), f32),
            pltpu.VMEM((16, 64), i32), pltpu.VMEM((16, 64), i32),
            pltpu.VMEM((16, 64), i32), pltpu.VMEM((16, 64), i32),
            pltpu.VMEM((16, 64 * HS), f32), pltpu.VMEM((16, 64 * HS), f32),
            pltpu.SemaphoreType.DMA, pltpu.SemaphoreType.DMA,
            pltpu.SemaphoreType.DMA, pltpu.SemaphoreType.DMA,
        ],
    )
    def scores(as_hbm, ad_hbm, src_hbm, dst_hbm, ex_hbm,
               as_v, ad_v, srcA, srcB, dstA, dstB, exA, exB,
               lsemA, lsemB, ssemA, ssemB):
        c = lax.axis_index("c")
        s = lax.axis_index("s")
        pltpu.sync_copy(as_hbm.at[c], as_v)
        pltpu.sync_copy(ad_hbm.at[c], ad_v)

        srcs = [srcA, srcB]
        dsts = [dstA, dstB]
        exs = [exA, exB]
        lsems = [lsemA, lsemB]
        ssems = [ssemA, ssemB]

        def issue_loads(g):
            b = g % 2
            base64 = s * (EPT // 64) + g * 16
            d1 = pltpu.async_copy(src_hbm.at[c, pl.ds(base64, 16)],
                                  srcs[b], lsems[b])
            d2 = pltpu.async_copy(dst_hbm.at[c, pl.ds(base64, 16)],
                                  dsts[b], lsems[b])
            return (d1, d2)

        lds = {0: issue_loads(0)}
        sts = {}
        for g in range(SUP):
            b = g % 2
            if g + 1 < SUP:
                lds[g + 1] = issue_loads(g + 1)
            lds[g][0].wait()
            lds[g][1].wait()
            if g >= 2:
                sts[g - 2].wait()
            src_v, dst_v, ex_v = srcs[b], dsts[b], exs[b]

            def row(r, _):
                for t in range(4):
                    sv = src_v[r, pl.ds(t * 16, 16)]
                    dv = dst_v[r, pl.ds(t * 16, 16)]
                    for h in range(HS):
                        a = (plsc.load_gather(as_v, [sv * HS + h]) +
                             plsc.load_gather(ad_v, [dv * HS + h]))
                        a = jnp.where(a > 0, a, 0.2 * a)
                        ex_v[r, pl.ds(h * 64 + t * 16, 16)] = jnp.exp(a)
                return 0

            lax.fori_loop(0, 16, row, 0)
            base64 = s * (EPT // 64) + g * 16
            sts[g] = pltpu.async_copy(
                ex_v, ex_hbm.at[c, pl.ds(base64, 16)], ssems[b])
        sts[SUP - 2].wait()
        sts[SUP - 1].wait()

    return scores


def _make_sc_msg(EP, HS, F):
    W = F + 16
    CH = F // HS
    EPT = EP // 16
    SUP = EPT // 1024
    RPT = RO // 16
    mesh = plsc.VectorSubcoreMesh(core_axis_name="c", subcore_axis_name="s")

    @functools.partial(
        pl.kernel,
        out_type=jax.ShapeDtypeStruct((2, RO, W), f32),
        mesh=mesh,
        compiler_params=_SC_PARAMS,
        scratch_types=[
            pltpu.VMEM((16, 64), i32),
            pltpu.VMEM((16, 64), i32),
            pltpu.VMEM((16, 64), i32),
            pltpu.VMEM((16, 64 * HS), f32),
            pltpu.VMEM((64, F), f32), pltpu.VMEM((64, F), f32),
            pltpu.VMEM((64, W), f32), pltpu.VMEM((64, W), f32),
            pltpu.VMEM_SHARED((RO, W), f32),
            pltpu.SemaphoreType.DMA,
            pltpu.SemaphoreType.DMA, pltpu.SemaphoreType.DMA,
            pltpu.SemaphoreType.DMA, pltpu.SemaphoreType.DMA,
        ],
    )
    def msg(hs_hbm, ex_hbm, src_hbm, dst_hbm, out_hbm,
            src_v, dst_v, srcg_v, ex_v, rowsA, rowsB, orowA, orowB,
            out_sh, lsem, gsemA, gsemB, ssemA, ssemB):
        c = lax.axis_index("c")
        s = lax.axis_index("s")

        zero16 = jnp.zeros((16,), f32)
        rows = [rowsA, rowsB]
        orows = [orowA, orowB]
        gsems = [gsemA, gsemB]
        ssems = [ssemA, ssemB]

        def zero_row(j, _):
            for t in range(W // 16):
                orowA[j, pl.ds(t * 16, 16)] = zero16
            return 0

        lax.fori_loop(0, 64, zero_row, 0)
        r0 = s * RPT
        for t in range(9):
            pltpu.sync_copy(orowA, out_sh.at[pl.ds(r0 + t * 64, 64)])
        pltpu.sync_copy(orowA.at[pl.ds(0, RPT - 576)],
                        out_sh.at[pl.ds(r0 + 576, RPT - 576)])
        plsc.subcore_barrier()

        offv = lax.broadcast(c * RT, (16,)).astype(i32)
        lane = lax.iota(i32, 16)

        def sup(g, _):
            base64 = s * (EPT // 64) + g * 16
            d1 = pltpu.async_copy(src_hbm.at[c, pl.ds(base64, 16)],
                                  src_v, lsem)
            d2 = pltpu.async_copy(dst_hbm.at[c, pl.ds(base64, 16)],
                                  dst_v, lsem)
            d3 = pltpu.async_copy(ex_hbm.at[c, pl.ds(base64, 16)],
                                  ex_v, lsem)
            d1.wait(); d2.wait(); d3.wait()

            def adj(r, _):
                for t in range(4):
                    srcg_v[r, pl.ds(t * 16, 16)] = (
                        src_v[r, pl.ds(t * 16, 16)] + offv)
                return 0

            lax.fori_loop(0, 16, adj, 0)

            gd = {0: pltpu.async_copy(hs_hbm.at[srcg_v.at[0]],
                                      rows[0], gsems[0])}
            sd = {}
            for j in range(16):
                b = j % 2
                if j + 1 < 16:
                    gd[j + 1] = pltpu.async_copy(
                        hs_hbm.at[srcg_v.at[j + 1]],
                        rows[(j + 1) % 2], gsems[(j + 1) % 2])
                gd[j].wait()
                rb = rows[b]
                ob = orows[b]

                def oct_(q, _):
                    g16 = (q // 2) * 16
                    evs = [ex_v[j, pl.ds(h * 64 + g16, 16)]
                           for h in range(HS)]
                    for k in range(8):
                        lanei = (q % 2) * 8 + k
                        bexs = [_bcast_lane(ev, lanei) for ev in evs]
                        e = q * 8 + k
                        for t in range(F // 16):
                            h = (t * 16) // CH
                            ob[e, pl.ds(t * 16, 16)] = (
                                rb[e, pl.ds(t * 16, 16)] * bexs[h])
                        tail = zero16
                        for h in range(HS):
                            tail = jnp.where(lane == h, bexs[h], tail)
                        ob[e, pl.ds(F, 16)] = tail
                    return 0

                lax.fori_loop(0, 8, oct_, 0)
                if j >= 1:
                    sd[j - 1].wait()
                sd[j] = pltpu.async_copy(
                    ob, out_sh.at[dst_v.at[j]], ssems[b], add=True)
            sd[15].wait()
            return 0

        lax.fori_loop(0, SUP, sup, 0)
        plsc.subcore_barrier()
        for t in range(9):
            pltpu.sync_copy(out_sh.at[pl.ds(r0 + t * 64, 64)], orowA)
            pltpu.sync_copy(orowA, out_hbm.at[c, pl.ds(r0 + t * 64, 64)])
        pltpu.sync_copy(out_sh.at[pl.ds(r0 + 576, RPT - 576)],
                        orowA.at[pl.ds(0, RPT - 576)])
        pltpu.sync_copy(orowA.at[pl.ds(0, RPT - 576)],
                        out_hbm.at[c, pl.ds(r0 + 576, RPT - 576)])

    return msg


_sc_scores_l0 = _make_sc_scores(EP=327680, HS=2)
_sc_scores_l1 = _make_sc_scores(EP=163840, HS=1)
_sc_msg_l0 = _make_sc_msg(EP=327680, HS=2, F=128)
_sc_msg_l1 = _make_sc_msg(EP=163840, HS=1, F=32)


def _sc_conv_l0(hs_t, as_t, ad_t, src, dst):
    ex = _sc_scores_l0(as_t, ad_t, src, dst)
    return _sc_msg_l0(hs_t, ex, src, dst)


def _sc_conv_l1(hs_t, as_t, ad_t, src, dst):
    ex = _sc_scores_l1(as_t, ad_t, src, dst)
    return _sc_msg_l1(hs_t, ex, src, dst)


def _pad_rows(x, val=0.0):
    return jnp.concatenate(
        [x, jnp.full((RT - N,) + x.shape[1:], val, x.dtype)], axis=0)


def _prep_edges_l0(ei):
    src = ei[0].astype(i32)
    dst = ei[1].astype(i32)
    pad = jnp.full((327680 - E,), N, i32)
    srcp = jnp.concatenate([src, pad]).reshape(327680 // 64, 64)
    dstp = jnp.concatenate([dst, pad]).reshape(327680 // 64, 64)
    return (jnp.broadcast_to(srcp[None], (2, 327680 // 64, 64)),
            jnp.broadcast_to(dstp[None], (2, 327680 // 64, 64)))


def _prep_edges_l1(ei):
    src = ei[0].astype(i32)
    dst = ei[1].astype(i32)
    half = E // 2
    pad = jnp.full((163840 - half,), N, i32)
    srcp = jnp.stack([jnp.concatenate([src[:half], pad]),
                      jnp.concatenate([src[half:], pad])])
    dstp = jnp.stack([jnp.concatenate([dst[:half], pad]),
                      jnp.concatenate([dst[half:], pad])])
    return (srcp.reshape(2, 163840 // 64, 64),
            dstp.reshape(2, 163840 // 64, 64))


def _score_tables_l0(a):
    ap = _pad_rows(a, NEG)
    return ap.reshape(RT, 2, 2).transpose(1, 0, 2).reshape(2, RT * 2)


def _score_tables_l1(a):
    ap = _pad_rows(a[:, None], NEG).reshape(RT)
    return jnp.broadcast_to(ap[None], (2, RT))


def _hs_table_l0(hs):
    hp = _pad_rows(hs)
    return jnp.concatenate([hp[:, :128], hp[:, 128:]], axis=0)


def _hs_table_l1(hs):
    hp = _pad_rows(hs)
    return jnp.concatenate([hp, hp], axis=0)


def _fold_l0(Wsrc, avec):
    return jnp.einsum('khc,hc->kh', Wsrc.reshape(64, 4, 64), avec)


def kernel(x_user, x_item, edge_index_u2i, edge_index_i2u,
           W_in_user, b_in_user, W_in_item, b_in_item,
           l0_u2i_Wsrc, l0_u2i_Wdst, l0_u2i_asrc, l0_u2i_adst, l0_u2i_b,
           l0_i2u_Wsrc, l0_i2u_Wdst, l0_i2u_asrc, l0_i2u_adst, l0_i2u_b,
           l1_u2i_Wsrc, l1_u2i_Wdst, l1_u2i_asrc, l1_u2i_adst, l1_u2i_b,
           l1_i2u_Wsrc, l1_i2u_Wdst, l1_i2u_asrc, l1_i2u_adst, l1_i2u_b):
    ws0_u2i = _fold_l0(l0_u2i_Wsrc, l0_u2i_asrc)
    wd0_u2i = _fold_l0(l0_u2i_Wdst, l0_u2i_adst)
    ws0_i2u = _fold_l0(l0_i2u_Wsrc, l0_i2u_asrc)
    wd0_i2u = _fold_l0(l0_i2u_Wdst, l0_i2u_adst)
    WbigU = jnp.concatenate([l0_u2i_Wsrc, ws0_u2i, wd0_i2u], axis=1)
    WbigI = jnp.concatenate([l0_i2u_Wsrc, ws0_i2u, wd0_u2i], axis=1)

    bigU = _tc_in_proj(x_user, W_in_user, b_in_user[None], WbigU)
    bigI = _tc_in_proj(x_item, W_in_item, b_in_item[None], WbigI)

    su0, du0 = _prep_edges_l0(edge_index_u2i)
    si0, di0 = _prep_edges_l0(edge_index_i2u)

    outU = _sc_conv_l0(_hs_table_l0(bigU[:, :256]),
                       _score_tables_l0(bigU[:, 256:260]),
                       _score_tables_l0(bigI[:, 260:264]),
                       su0, du0)
    outI = _sc_conv_l0(_hs_table_l0(bigI[:, :256]),
                       _score_tables_l0(bigI[:, 256:260]),
                       _score_tables_l0(bigU[:, 260:264]),
                       si0, di0)

    num_i = jnp.concatenate([outU[0, :N, :128], outU[1, :N, :128]], axis=1)
    den_i = jnp.concatenate([outU[0, :N, 128:130], outU[1, :N, 128:130]], axis=1)
    num_u = jnp.concatenate([outI[0, :N, :128], outI[1, :N, :128]], axis=1)
    den_u = jnp.concatenate([outI[0, :N, 128:130], outI[1, :N, 128:130]], axis=1)

    ws1_u2i = l1_u2i_Wsrc @ l1_u2i_asrc[0]
    wd1_u2i = l1_u2i_Wdst @ l1_u2i_adst[0]
    ws1_i2u = l1_i2u_Wsrc @ l1_i2u_asrc[0]
    wd1_i2u = l1_i2u_Wdst @ l1_i2u_adst[0]
    W1bigU = jnp.concatenate(
        [l1_u2i_Wsrc, ws1_u2i[:, None], wd1_i2u[:, None]], axis=1)
    W1bigI = jnp.concatenate(
        [l1_i2u_Wsrc, ws1_i2u[:, None], wd1_u2i[:, None]], axis=1)

    big1U = _tc_mid(num_u, den_u, l0_i2u_b[None], W1bigU)
    big1I = _tc_mid(num_i, den_i, l0_u2i_b[None], W1bigI)

    su1, du1 = _prep_edges_l1(edge_index_u2i)
    si1, di1 = _prep_edges_l1(edge_index_i2u)

    out1U = _sc_conv_l1(_hs_table_l1(big1U[:, :32]),
                        _score_tables_l1(big1U[:, 32]),
                        _score_tables_l1(big1I[:, 33]),
                        su1, du1)
    out1I = _sc_conv_l1(_hs_table_l1(big1I[:, :32]),
                        _score_tables_l1(big1I[:, 32]),
                        _score_tables_l1(big1U[:, 33]),
                        si1, di1)

    hi2 = _tc_final(out1U[0, :N, :33], out1U[1, :N, :33], l1_u2i_b[None])
    hu2 = _tc_final(out1I[0, :N, :33], out1I[1, :N, :33], l1_i2u_b[None])
    return hu2, hi2


# speedup vs baseline: 48.9140x; 1.3505x over previous
import functools

import jax
import jax.numpy as jnp
from jax import lax
from jax.experimental import pallas as pl
from jax.experimental.pallas import tpu as pltpu
from jax.experimental.pallas import tpu_sc as plsc

N = 10000
E = 320000
RT = 10016
RO = 10016
NEG = -1.0e30

f32 = jnp.float32
i32 = jnp.int32

_SC_PARAMS = pltpu.CompilerParams(
    needs_layout_passes=False, use_tc_tiling_on_sc=False)


def _tc_in_proj(x, W1, b1, Wbig):
    B = 400
    K2 = Wbig.shape[1]

    def body(x_r, w1_r, b1_r, wb_r, o_r):
        h = jnp.dot(x_r[...], w1_r[...], preferred_element_type=f32) + b1_r[...]
        h = jnp.where(h > 0, h, jnp.exp(jnp.minimum(h, 0.0)) - 1.0)
        o_r[...] = jnp.dot(h, wb_r[...], preferred_element_type=f32)

    return pl.pallas_call(
        body,
        grid=(N // B,),
        in_specs=[
            pl.BlockSpec((B, x.shape[1]), lambda i: (i, 0)),
            pl.BlockSpec(W1.shape, lambda i: (0, 0)),
            pl.BlockSpec(b1.shape, lambda i: (0, 0)),
            pl.BlockSpec(Wbig.shape, lambda i: (0, 0)),
        ],
        out_specs=pl.BlockSpec((B, K2), lambda i: (i, 0)),
        out_shape=jax.ShapeDtypeStruct((N, K2), f32),
    )(x, W1, b1, Wbig)


def _tc_mid(num, den, b, Wbig):
    B = 400
    K2 = Wbig.shape[1]

    def body(n_r, d_r, b_r, wb_r, o_r):
        d = d_r[...]
        drep = jnp.concatenate(
            [jnp.broadcast_to(d[:, h:h + 1], (B, 64)) for h in range(4)], axis=1)
        t = n_r[...] / (drep + 1e-16) + b_r[...]
        t = jnp.where(t > 0, t, jnp.exp(jnp.minimum(t, 0.0)) - 1.0)
        o_r[...] = jnp.dot(t, wb_r[...], preferred_element_type=f32)

    return pl.pallas_call(
        body,
        grid=(N // B,),
        in_specs=[
            pl.BlockSpec((B, 256), lambda i: (i, 0)),
            pl.BlockSpec((B, 4), lambda i: (i, 0)),
            pl.BlockSpec(b.shape, lambda i: (0, 0)),
            pl.BlockSpec(Wbig.shape, lambda i: (0, 0)),
        ],
        out_specs=pl.BlockSpec((B, K2), lambda i: (i, 0)),
        out_shape=jax.ShapeDtypeStruct((N, K2), f32),
    )(num, den, b, Wbig)


def _tc_final(t0, t1, b):
    B = 1000

    def body(a_r, c_r, b_r, o_r):
        s = a_r[...] + c_r[...]
        o_r[...] = s[:, :32] / (s[:, 32:33] + 1e-16) + b_r[...]

    return pl.pallas_call(
        body,
        grid=(N // B,),
        in_specs=[
            pl.BlockSpec((B, 33), lambda i: (i, 0)),
            pl.BlockSpec((B, 33), lambda i: (i, 0)),
            pl.BlockSpec(b.shape, lambda i: (0, 0)),
        ],
        out_specs=pl.BlockSpec((B, 32), lambda i: (i, 0)),
        out_shape=jax.ShapeDtypeStruct((N, 32), f32),
    )(t0, t1, b)


_GDN = lax.GatherDimensionNumbers(
    offset_dims=(), collapsed_slice_dims=(0,), start_index_map=(0,))


def _bcast_lane(ev, lanei):
    sel = lax.broadcast(lanei, (16,)).astype(i32).reshape(16, 1)
    return lax.gather(ev, sel, dimension_numbers=_GDN, slice_sizes=(1,),
                      mode=lax.GatherScatterMode.PROMISE_IN_BOUNDS)


def _make_sc_scores(EP, HS):
    EPT = EP // 16
    SUP = EPT // 1024
    mesh = plsc.VectorSubcoreMesh(core_axis_name="c", subcore_axis_name="s")

    @functools.partial(
        pl.kernel,
        out_type=jax.ShapeDtypeStruct((2, EP // 64, 64 * HS), f32),
        mesh=mesh,
        compiler_params=_SC_PARAMS,
        scratch_types=[
            pltpu.VMEM((RT * HS,), f32),
            pltpu.VMEM((RT * HS,), f32),
            pltpu.VMEM((16, 64), i32), pltpu.VMEM((16, 64), i32),
            pltpu.VMEM((16, 64), i32), pltpu.VMEM((16, 64), i32),
            pltpu.VMEM((16, 64 * HS), f32), pltpu.VMEM((16, 64 * HS), f32),
            pltpu.SemaphoreType.DMA, pltpu.SemaphoreType.DMA,
            pltpu.SemaphoreType.DMA, pltpu.SemaphoreType.DMA,
        ],
    )
    def scores(as_hbm, ad_hbm, src_hbm, dst_hbm, ex_hbm,
               as_v, ad_v, srcA, srcB, dstA, dstB, exA, exB,
               lsemA, lsemB, ssemA, ssemB):
        c = lax.axis_index("c")
        s = lax.axis_index("s")
        pltpu.sync_copy(as_hbm.at[c], as_v)
        pltpu.sync_copy(ad_hbm.at[c], ad_v)

        srcs = [srcA, srcB]
        dsts = [dstA, dstB]
        exs = [exA, exB]
        lsems = [lsemA, lsemB]
        ssems = [ssemA, ssemB]

        def issue_loads(g):
            b = g % 2
            base64 = s * (EPT // 64) + g * 16
            d1 = pltpu.async_copy(src_hbm.at[c, pl.ds(base64, 16)],
                                  srcs[b], lsems[b])
            d2 = pltpu.async_copy(dst_hbm.at[c, pl.ds(base64, 16)],
                                  dsts[b], lsems[b])
            return (d1, d2)

        lds = {0: issue_loads(0)}
        sts = {}
        for g in range(SUP):
            b = g % 2
            if g + 1 < SUP:
                lds[g + 1] = issue_loads(g + 1)
            lds[g][0].wait()
            lds[g][1].wait()
            if g >= 2:
                sts[g - 2].wait()
            src_v, dst_v, ex_v = srcs[b], dsts[b], exs[b]

            def row(r, _):
                for t in range(4):
                    sv = src_v[r, pl.ds(t * 16, 16)]
                    dv = dst_v[r, pl.ds(t * 16, 16)]
                    for h in range(HS):
                        a = (plsc.load_gather(as_v, [sv * HS + h]) +
                             plsc.load_gather(ad_v, [dv * HS + h]))
                        a = jnp.where(a > 0, a, 0.2 * a)
                        ex_v[r, pl.ds(h * 64 + t * 16, 16)] = jnp.exp(a)
                return 0

            lax.fori_loop(0, 16, row, 0)
            base64 = s * (EPT // 64) + g * 16
            sts[g] = pltpu.async_copy(
                ex_v, ex_hbm.at[c, pl.ds(base64, 16)], ssems[b])
        sts[SUP - 2].wait()
        sts[SUP - 1].wait()

    return scores


def _make_sc_msg(EP, HS, F):
    W = F + 16
    CH = F // HS
    EPT = EP // 16
    SUP = EPT // 1024
    RPT = RO // 16
    mesh = plsc.VectorSubcoreMesh(core_axis_name="c", subcore_axis_name="s")

    @functools.partial(
        pl.kernel,
        out_type=jax.ShapeDtypeStruct((2, RO, W), f32),
        mesh=mesh,
        compiler_params=_SC_PARAMS,
        scratch_types=[
            pltpu.VMEM((16, 64), i32),
            pltpu.VMEM((16, 64), i32),
            pltpu.VMEM((16, 64), i32),
            pltpu.VMEM((16, 64 * HS), f32),
            pltpu.VMEM((64, F), f32), pltpu.VMEM((64, F), f32),
            pltpu.VMEM((64, W), f32), pltpu.VMEM((64, W), f32),
            pltpu.VMEM_SHARED((RO, W), f32),
            pltpu.SemaphoreType.DMA,
            pltpu.SemaphoreType.DMA, pltpu.SemaphoreType.DMA,
            pltpu.SemaphoreType.DMA, pltpu.SemaphoreType.DMA,
        ],
    )
    def msg(hs_hbm, ex_hbm, src_hbm, dst_hbm, out_hbm,
            src_v, dst_v, srcg_v, ex_v, rowsA, rowsB, orowA, orowB,
            out_sh, lsem, gsemA, gsemB, ssemA, ssemB):
        c = lax.axis_index("c")
        s = lax.axis_index("s")

        zero16 = jnp.zeros((16,), f32)
        rows = [rowsA, rowsB]
        orows = [orowA, orowB]
        gsems = [gsemA, gsemB]
        ssems = [ssemA, ssemB]

        def zero_row(j, _):
            for t in range(W // 16):
                orowA[j, pl.ds(t * 16, 16)] = zero16
            return 0

        lax.fori_loop(0, 64, zero_row, 0)
        r0 = s * RPT
        for t in range(9):
            pltpu.sync_copy(orowA, out_sh.at[pl.ds(r0 + t * 64, 64)])
        pltpu.sync_copy(orowA.at[pl.ds(0, RPT - 576)],
                        out_sh.at[pl.ds(r0 + 576, RPT - 576)])
        plsc.subcore_barrier()

        offv = lax.broadcast(c * RT, (16,)).astype(i32)
        lane = lax.iota(i32, 16)

        def sup(g, _):
            base64 = s * (EPT // 64) + g * 16
            d1 = pltpu.async_copy(src_hbm.at[c, pl.ds(base64, 16)],
                                  src_v, lsem)
            d2 = pltpu.async_copy(dst_hbm.at[c, pl.ds(base64, 16)],
                                  dst_v, lsem)
            d3 = pltpu.async_copy(ex_hbm.at[c, pl.ds(base64, 16)],
                                  ex_v, lsem)
            d1.wait(); d2.wait(); d3.wait()

            def adj(r, _):
                for t in range(4):
                    srcg_v[r, pl.ds(t * 16, 16)] = (
                        src_v[r, pl.ds(t * 16, 16)] + offv)
                return 0

            lax.fori_loop(0, 16, adj, 0)

            gd = {0: pltpu.async_copy(hs_hbm.at[srcg_v.at[0]],
                                      rows[0], gsems[0])}
            sd = {}
            for j in range(16):
                b = j % 2
                if j + 1 < 16:
                    gd[j + 1] = pltpu.async_copy(
                        hs_hbm.at[srcg_v.at[j + 1]],
                        rows[(j + 1) % 2], gsems[(j + 1) % 2])
                gd[j].wait()
                rb = rows[b]
                ob = orows[b]

                def oct_(q, _):
                    g16 = (q // 2) * 16
                    evs = [ex_v[j, pl.ds(h * 64 + g16, 16)]
                           for h in range(HS)]
                    NT = F // 16
                    for k in range(8):
                        lanei = (q % 2) * 8 + k
                        bexs = [_bcast_lane(ev, lanei) for ev in evs]
                        e = q * 8 + k
                        vals = [rb[e, pl.ds(t * 16, 16)] for t in range(NT)]
                        prods = [vals[t] * bexs[(t * 16) // CH]
                                 for t in range(NT)]
                        tail = zero16
                        for h in range(HS):
                            tail = jnp.where(lane == h, bexs[h], tail)
                        for t in range(NT):
                            ob[e, pl.ds(t * 16, 16)] = prods[t]
                        ob[e, pl.ds(F, 16)] = tail
                    return 0

                lax.fori_loop(0, 8, oct_, 0)
                if j >= 1:
                    sd[j - 1].wait()
                sd[j] = pltpu.async_copy(
                    ob, out_sh.at[dst_v.at[j]], ssems[b], add=True)
            sd[15].wait()
            return 0

        lax.fori_loop(0, SUP, sup, 0)
        plsc.subcore_barrier()
        for t in range(9):
            pltpu.sync_copy(out_sh.at[pl.ds(r0 + t * 64, 64)], orowA)
            pltpu.sync_copy(orowA, out_hbm.at[c, pl.ds(r0 + t * 64, 64)])
        pltpu.sync_copy(out_sh.at[pl.ds(r0 + 576, RPT - 576)],
                        orowA.at[pl.ds(0, RPT - 576)])
        pltpu.sync_copy(orowA.at[pl.ds(0, RPT - 576)],
                        out_hbm.at[c, pl.ds(r0 + 576, RPT - 576)])

    return msg


_sc_scores_l0 = _make_sc_scores(EP=327680, HS=2)
_sc_scores_l1 = _make_sc_scores(EP=163840, HS=1)
_sc_msg_l0 = _make_sc_msg(EP=327680, HS=2, F=128)
_sc_msg_l1 = _make_sc_msg(EP=163840, HS=1, F=32)


def _sc_conv_l0(hs_t, as_t, ad_t, src, dst):
    ex = _sc_scores_l0(as_t, ad_t, src, dst)
    return _sc_msg_l0(hs_t, ex, src, dst)


def _sc_conv_l1(hs_t, as_t, ad_t, src, dst):
    ex = _sc_scores_l1(as_t, ad_t, src, dst)
    return _sc_msg_l1(hs_t, ex, src, dst)


def _pad_rows(x, val=0.0):
    return jnp.concatenate(
        [x, jnp.full((RT - N,) + x.shape[1:], val, x.dtype)], axis=0)


def _prep_edges_l0(ei):
    src = ei[0].astype(i32)
    dst = ei[1].astype(i32)
    pad = jnp.full((327680 - E,), N, i32)
    srcp = jnp.concatenate([src, pad]).reshape(327680 // 64, 64)
    dstp = jnp.concatenate([dst, pad]).reshape(327680 // 64, 64)
    return (jnp.broadcast_to(srcp[None], (2, 327680 // 64, 64)),
            jnp.broadcast_to(dstp[None], (2, 327680 // 64, 64)))


def _prep_edges_l1(ei):
    src = ei[0].astype(i32)
    dst = ei[1].astype(i32)
    half = E // 2
    pad = jnp.full((163840 - half,), N, i32)
    srcp = jnp.stack([jnp.concatenate([src[:half], pad]),
                      jnp.concatenate([src[half:], pad])])
    dstp = jnp.stack([jnp.concatenate([dst[:half], pad]),
                      jnp.concatenate([dst[half:], pad])])
    return (srcp.reshape(2, 163840 // 64, 64),
            dstp.reshape(2, 163840 // 64, 64))


def _score_tables_l0(a):
    ap = _pad_rows(a, NEG)
    return ap.reshape(RT, 2, 2).transpose(1, 0, 2).reshape(2, RT * 2)


def _score_tables_l1(a):
    ap = _pad_rows(a[:, None], NEG).reshape(RT)
    return jnp.broadcast_to(ap[None], (2, RT))


def _hs_table_l0(hs):
    hp = _pad_rows(hs)
    return jnp.concatenate([hp[:, :128], hp[:, 128:]], axis=0)


def _hs_table_l1(hs):
    hp = _pad_rows(hs)
    return jnp.concatenate([hp, hp], axis=0)


def _fold_l0(Wsrc, avec):
    return jnp.einsum('khc,hc->kh', Wsrc.reshape(64, 4, 64), avec)


def kernel(x_user, x_item, edge_index_u2i, edge_index_i2u,
           W_in_user, b_in_user, W_in_item, b_in_item,
           l0_u2i_Wsrc, l0_u2i_Wdst, l0_u2i_asrc, l0_u2i_adst, l0_u2i_b,
           l0_i2u_Wsrc, l0_i2u_Wdst, l0_i2u_asrc, l0_i2u_adst, l0_i2u_b,
           l1_u2i_Wsrc, l1_u2i_Wdst, l1_u2i_asrc, l1_u2i_adst, l1_u2i_b,
           l1_i2u_Wsrc, l1_i2u_Wdst, l1_i2u_asrc, l1_i2u_adst, l1_i2u_b):
    ws0_u2i = _fold_l0(l0_u2i_Wsrc, l0_u2i_asrc)
    wd0_u2i = _fold_l0(l0_u2i_Wdst, l0_u2i_adst)
    ws0_i2u = _fold_l0(l0_i2u_Wsrc, l0_i2u_asrc)
    wd0_i2u = _fold_l0(l0_i2u_Wdst, l0_i2u_adst)
    WbigU = jnp.concatenate([l0_u2i_Wsrc, ws0_u2i, wd0_i2u], axis=1)
    WbigI = jnp.concatenate([l0_i2u_Wsrc, ws0_i2u, wd0_u2i], axis=1)

    bigU = _tc_in_proj(x_user, W_in_user, b_in_user[None], WbigU)
    bigI = _tc_in_proj(x_item, W_in_item, b_in_item[None], WbigI)

    su0, du0 = _prep_edges_l0(edge_index_u2i)
    si0, di0 = _prep_edges_l0(edge_index_i2u)

    outU = _sc_conv_l0(_hs_table_l0(bigU[:, :256]),
                       _score_tables_l0(bigU[:, 256:260]),
                       _score_tables_l0(bigI[:, 260:264]),
                       su0, du0)
    outI = _sc_conv_l0(_hs_table_l0(bigI[:, :256]),
                       _score_tables_l0(bigI[:, 256:260]),
                       _score_tables_l0(bigU[:, 260:264]),
                       si0, di0)

    num_i = jnp.concatenate([outU[0, :N, :128], outU[1, :N, :128]], axis=1)
    den_i = jnp.concatenate([outU[0, :N, 128:130], outU[1, :N, 128:130]], axis=1)
    num_u = jnp.concatenate([outI[0, :N, :128], outI[1, :N, :128]], axis=1)
    den_u = jnp.concatenate([outI[0, :N, 128:130], outI[1, :N, 128:130]], axis=1)

    ws1_u2i = l1_u2i_Wsrc @ l1_u2i_asrc[0]
    wd1_u2i = l1_u2i_Wdst @ l1_u2i_adst[0]
    ws1_i2u = l1_i2u_Wsrc @ l1_i2u_asrc[0]
    wd1_i2u = l1_i2u_Wdst @ l1_i2u_adst[0]
    W1bigU = jnp.concatenate(
        [l1_u2i_Wsrc, ws1_u2i[:, None], wd1_i2u[:, None]], axis=1)
    W1bigI = jnp.concatenate(
        [l1_i2u_Wsrc, ws1_i2u[:, None], wd1_u2i[:, None]], axis=1)

    big1U = _tc_mid(num_u, den_u, l0_i2u_b[None], W1bigU)
    big1I = _tc_mid(num_i, den_i, l0_u2i_b[None], W1bigI)

    su1, du1 = _prep_edges_l1(edge_index_u2i)
    si1, di1 = _prep_edges_l1(edge_index_i2u)

    out1U = _sc_conv_l1(_hs_table_l1(big1U[:, :32]),
                        _score_tables_l1(big1U[:, 32]),
                        _score_tables_l1(big1I[:, 33]),
                        su1, du1)
    out1I = _sc_conv_l1(_hs_table_l1(big1I[:, :32]),
                        _score_tables_l1(big1I[:, 32]),
                        _score_tables_l1(big1U[:, 33]),
                        si1, di1)

    hi2 = _tc_final(out1U[0, :N, :33], out1U[1, :N, :33], l1_u2i_b[None])
    hu2 = _tc_final(out1I[0, :N, :33], out1I[1, :N, :33], l1_i2u_b[None])
    return hu2, hi2


# speedup vs baseline: 49.6498x; 1.0150x over previous
import functools

import jax
import jax.numpy as jnp
from jax import lax
from jax.experimental import pallas as pl
from jax.experimental.pallas import tpu as pltpu
from jax.experimental.pallas import tpu_sc as plsc

N = 10000
E = 320000
RT = 10016
RO = 10016
NEG = -1.0e30

f32 = jnp.float32
i32 = jnp.int32

_SC_PARAMS = pltpu.CompilerParams(
    needs_layout_passes=False, use_tc_tiling_on_sc=False)


def _tc_in_proj(x, W1, b1, Wbig):
    B = 400
    K2 = Wbig.shape[1]

    def body(x_r, w1_r, b1_r, wb_r, o_r):
        h = jnp.dot(x_r[...], w1_r[...], preferred_element_type=f32) + b1_r[...]
        h = jnp.where(h > 0, h, jnp.exp(jnp.minimum(h, 0.0)) - 1.0)
        o_r[...] = jnp.dot(h, wb_r[...], preferred_element_type=f32)

    return pl.pallas_call(
        body,
        grid=(N // B,),
        in_specs=[
            pl.BlockSpec((B, x.shape[1]), lambda i: (i, 0)),
            pl.BlockSpec(W1.shape, lambda i: (0, 0)),
            pl.BlockSpec(b1.shape, lambda i: (0, 0)),
            pl.BlockSpec(Wbig.shape, lambda i: (0, 0)),
        ],
        out_specs=pl.BlockSpec((B, K2), lambda i: (i, 0)),
        out_shape=jax.ShapeDtypeStruct((N, K2), f32),
    )(x, W1, b1, Wbig)


def _tc_mid(num, den, b, Wbig):
    B = 400
    K2 = Wbig.shape[1]

    def body(n_r, d_r, b_r, wb_r, o_r):
        d = d_r[...]
        drep = jnp.concatenate(
            [jnp.broadcast_to(d[:, h:h + 1], (B, 64)) for h in range(4)], axis=1)
        t = n_r[...] / (drep + 1e-16) + b_r[...]
        t = jnp.where(t > 0, t, jnp.exp(jnp.minimum(t, 0.0)) - 1.0)
        o_r[...] = jnp.dot(t, wb_r[...], preferred_element_type=f32)

    return pl.pallas_call(
        body,
        grid=(N // B,),
        in_specs=[
            pl.BlockSpec((B, 256), lambda i: (i, 0)),
            pl.BlockSpec((B, 4), lambda i: (i, 0)),
            pl.BlockSpec(b.shape, lambda i: (0, 0)),
            pl.BlockSpec(Wbig.shape, lambda i: (0, 0)),
        ],
        out_specs=pl.BlockSpec((B, K2), lambda i: (i, 0)),
        out_shape=jax.ShapeDtypeStruct((N, K2), f32),
    )(num, den, b, Wbig)


def _tc_final(t0, t1, b):
    B = 1000

    def body(a_r, c_r, b_r, o_r):
        s = a_r[...] + c_r[...]
        o_r[...] = s[:, :32] / (s[:, 32:33] + 1e-16) + b_r[...]

    return pl.pallas_call(
        body,
        grid=(N // B,),
        in_specs=[
            pl.BlockSpec((B, 33), lambda i: (i, 0)),
            pl.BlockSpec((B, 33), lambda i: (i, 0)),
            pl.BlockSpec(b.shape, lambda i: (0, 0)),
        ],
        out_specs=pl.BlockSpec((B, 32), lambda i: (i, 0)),
        out_shape=jax.ShapeDtypeStruct((N, 32), f32),
    )(t0, t1, b)


_GDN = lax.GatherDimensionNumbers(
    offset_dims=(), collapsed_slice_dims=(0,), start_index_map=(0,))


def _bcast_lane(ev, lanei):
    sel = lax.broadcast(lanei, (16,)).astype(i32).reshape(16, 1)
    return lax.gather(ev, sel, dimension_numbers=_GDN, slice_sizes=(1,),
                      mode=lax.GatherScatterMode.PROMISE_IN_BOUNDS)


def _make_sc_scores(EP, HS):
    EPT = EP // 16
    SUP = EPT // 1024
    mesh = plsc.VectorSubcoreMesh(core_axis_name="c", subcore_axis_name="s")

    @functools.partial(
        pl.kernel,
        out_type=jax.ShapeDtypeStruct((2, EP // 64, 64 * HS), f32),
        mesh=mesh,
        compiler_params=_SC_PARAMS,
        scratch_types=[
            pltpu.VMEM((RT * HS,), f32),
            pltpu.VMEM((RT * HS,), f32),
            pltpu.VMEM((16, 64), i32), pltpu.VMEM((16, 64), i32),
            pltpu.VMEM((16, 64), i32), pltpu.VMEM((16, 64), i32),
            pltpu.VMEM((16, 64 * HS), f32), pltpu.VMEM((16, 64 * HS), f32),
            pltpu.SemaphoreType.DMA, pltpu.SemaphoreType.DMA,
            pltpu.SemaphoreType.DMA, pltpu.SemaphoreType.DMA,
        ],
    )
    def scores(as_hbm, ad_hbm, src_hbm, dst_hbm, ex_hbm,
               as_v, ad_v, srcA, srcB, dstA, dstB, exA, exB,
               lsemA, lsemB, ssemA, ssemB):
        c = lax.axis_index("c")
        s = lax.axis_index("s")
        pltpu.sync_copy(as_hbm.at[c], as_v)
        pltpu.sync_copy(ad_hbm.at[c], ad_v)

        srcs = [srcA, srcB]
        dsts = [dstA, dstB]
        exs = [exA, exB]
        lsems = [lsemA, lsemB]
        ssems = [ssemA, ssemB]

        def issue_loads(g):
            b = g % 2
            base64 = s * (EPT // 64) + g * 16
            d1 = pltpu.async_copy(src_hbm.at[c, pl.ds(base64, 16)],
                                  srcs[b], lsems[b])
            d2 = pltpu.async_copy(dst_hbm.at[c, pl.ds(base64, 16)],
                                  dsts[b], lsems[b])
            return (d1, d2)

        lds = {0: issue_loads(0)}
        sts = {}
        for g in range(SUP):
            b = g % 2
            if g + 1 < SUP:
                lds[g + 1] = issue_loads(g + 1)
            lds[g][0].wait()
            lds[g][1].wait()
            if g >= 2:
                sts[g - 2].wait()
            src_v, dst_v, ex_v = srcs[b], dsts[b], exs[b]

            def row(r, _):
                svs = [src_v[r, pl.ds(t * 16, 16)] for t in range(4)]
                dvs = [dst_v[r, pl.ds(t * 16, 16)] for t in range(4)]
                ga = [plsc.load_gather(as_v, [svs[t] * HS + h])
                      for t in range(4) for h in range(HS)]
                gd_ = [plsc.load_gather(ad_v, [dvs[t] * HS + h])
                       for t in range(4) for h in range(HS)]
                al = [ga[i] + gd_[i] for i in range(4 * HS)]
                al = [jnp.where(a > 0, a, 0.2 * a) for a in al]
                ex = [jnp.exp(a) for a in al]
                for t in range(4):
                    for h in range(HS):
                        ex_v[r, pl.ds(h * 64 + t * 16, 16)] = ex[t * HS + h]
                return 0

            lax.fori_loop(0, 16, row, 0)
            base64 = s * (EPT // 64) + g * 16
            sts[g] = pltpu.async_copy(
                ex_v, ex_hbm.at[c, pl.ds(base64, 16)], ssems[b])
        sts[SUP - 2].wait()
        sts[SUP - 1].wait()

    return scores


def _make_sc_msg(EP, HS, F):
    W = F + 16
    CH = F // HS
    EPT = EP // 16
    SUP = EPT // 1024
    RPT = RO // 16
    mesh = plsc.VectorSubcoreMesh(core_axis_name="c", subcore_axis_name="s")

    @functools.partial(
        pl.kernel,
        out_type=jax.ShapeDtypeStruct((2, RO, W), f32),
        mesh=mesh,
        compiler_params=_SC_PARAMS,
        scratch_types=[
            pltpu.VMEM((16, 64), i32),
            pltpu.VMEM((16, 64), i32),
            pltpu.VMEM((16, 64), i32),
            pltpu.VMEM((16, 64 * HS), f32),
            pltpu.VMEM((64, F), f32), pltpu.VMEM((64, F), f32),
            pltpu.VMEM((64, W), f32), pltpu.VMEM((64, W), f32),
            pltpu.VMEM_SHARED((RO, W), f32),
            pltpu.SemaphoreType.DMA,
            pltpu.SemaphoreType.DMA, pltpu.SemaphoreType.DMA,
            pltpu.SemaphoreType.DMA, pltpu.SemaphoreType.DMA,
        ],
    )
    def msg(hs_hbm, ex_hbm, src_hbm, dst_hbm, out_hbm,
            src_v, dst_v, srcg_v, ex_v, rowsA, rowsB, orowA, orowB,
            out_sh, lsem, gsemA, gsemB, ssemA, ssemB):
        c = lax.axis_index("c")
        s = lax.axis_index("s")

        zero16 = jnp.zeros((16,), f32)
        rows = [rowsA, rowsB]
        orows = [orowA, orowB]
        gsems = [gsemA, gsemB]
        ssems = [ssemA, ssemB]

        def zero_row(j, _):
            for t in range(W // 16):
                orowA[j, pl.ds(t * 16, 16)] = zero16
            return 0

        lax.fori_loop(0, 64, zero_row, 0)
        r0 = s * RPT
        for t in range(9):
            pltpu.sync_copy(orowA, out_sh.at[pl.ds(r0 + t * 64, 64)])
        pltpu.sync_copy(orowA.at[pl.ds(0, RPT - 576)],
                        out_sh.at[pl.ds(r0 + 576, RPT - 576)])
        plsc.subcore_barrier()

        offv = lax.broadcast(c * RT, (16,)).astype(i32)
        lane = lax.iota(i32, 16)

        def sup(g, _):
            base64 = s * (EPT // 64) + g * 16
            d1 = pltpu.async_copy(src_hbm.at[c, pl.ds(base64, 16)],
                                  src_v, lsem)
            d2 = pltpu.async_copy(dst_hbm.at[c, pl.ds(base64, 16)],
                                  dst_v, lsem)
            d3 = pltpu.async_copy(ex_hbm.at[c, pl.ds(base64, 16)],
                                  ex_v, lsem)
            d1.wait(); d2.wait(); d3.wait()

            def adj(r, _):
                for t in range(4):
                    srcg_v[r, pl.ds(t * 16, 16)] = (
                        src_v[r, pl.ds(t * 16, 16)] + offv)
                return 0

            lax.fori_loop(0, 16, adj, 0)

            gd = {0: pltpu.async_copy(hs_hbm.at[srcg_v.at[0]],
                                      rows[0], gsems[0])}
            sd = {}
            for j in range(16):
                b = j % 2
                if j + 1 < 16:
                    gd[j + 1] = pltpu.async_copy(
                        hs_hbm.at[srcg_v.at[j + 1]],
                        rows[(j + 1) % 2], gsems[(j + 1) % 2])
                gd[j].wait()
                rb = rows[b]
                ob = orows[b]

                def oct_(q, _):
                    g16 = (q // 2) * 16
                    evs = [ex_v[j, pl.ds(h * 64 + g16, 16)]
                           for h in range(HS)]
                    NT = F // 16
                    for k in range(8):
                        lanei = (q % 2) * 8 + k
                        bexs = [_bcast_lane(ev, lanei) for ev in evs]
                        e = q * 8 + k
                        vals = [rb[e, pl.ds(t * 16, 16)] for t in range(NT)]
                        prods = [vals[t] * bexs[(t * 16) // CH]
                                 for t in range(NT)]
                        tail = zero16
                        for h in range(HS):
                            tail = jnp.where(lane == h, bexs[h], tail)
                        for t in range(NT):
                            ob[e, pl.ds(t * 16, 16)] = prods[t]
                        ob[e, pl.ds(F, 16)] = tail
                    return 0

                lax.fori_loop(0, 8, oct_, 0)
                if j >= 1:
                    sd[j - 1].wait()
                sd[j] = pltpu.async_copy(
                    ob, out_sh.at[dst_v.at[j]], ssems[b], add=True)
            sd[15].wait()
            return 0

        lax.fori_loop(0, SUP, sup, 0)
        plsc.subcore_barrier()
        for t in range(9):
            pltpu.sync_copy(out_sh.at[pl.ds(r0 + t * 64, 64)], orowA)
            pltpu.sync_copy(orowA, out_hbm.at[c, pl.ds(r0 + t * 64, 64)])
        pltpu.sync_copy(out_sh.at[pl.ds(r0 + 576, RPT - 576)],
                        orowA.at[pl.ds(0, RPT - 576)])
        pltpu.sync_copy(orowA.at[pl.ds(0, RPT - 576)],
                        out_hbm.at[c, pl.ds(r0 + 576, RPT - 576)])

    return msg


_sc_scores_l0 = _make_sc_scores(EP=327680, HS=2)
_sc_scores_l1 = _make_sc_scores(EP=163840, HS=1)
_sc_msg_l0 = _make_sc_msg(EP=327680, HS=2, F=128)
_sc_msg_l1 = _make_sc_msg(EP=163840, HS=1, F=32)


def _sc_conv_l0(hs_t, as_t, ad_t, src, dst):
    ex = _sc_scores_l0(as_t, ad_t, src, dst)
    return _sc_msg_l0(hs_t, ex, src, dst)


def _sc_conv_l1(hs_t, as_t, ad_t, src, dst):
    ex = _sc_scores_l1(as_t, ad_t, src, dst)
    return _sc_msg_l1(hs_t, ex, src, dst)


def _pad_rows(x, val=0.0):
    return jnp.concatenate(
        [x, jnp.full((RT - N,) + x.shape[1:], val, x.dtype)], axis=0)


def _prep_edges_l0(ei):
    src = ei[0].astype(i32)
    dst = ei[1].astype(i32)
    pad = jnp.full((327680 - E,), N, i32)
    srcp = jnp.concatenate([src, pad]).reshape(327680 // 64, 64)
    dstp = jnp.concatenate([dst, pad]).reshape(327680 // 64, 64)
    return (jnp.broadcast_to(srcp[None], (2, 327680 // 64, 64)),
            jnp.broadcast_to(dstp[None], (2, 327680 // 64, 64)))


def _prep_edges_l1(ei):
    src = ei[0].astype(i32)
    dst = ei[1].astype(i32)
    half = E // 2
    pad = jnp.full((163840 - half,), N, i32)
    srcp = jnp.stack([jnp.concatenate([src[:half], pad]),
                      jnp.concatenate([src[half:], pad])])
    dstp = jnp.stack([jnp.concatenate([dst[:half], pad]),
                      jnp.concatenate([dst[half:], pad])])
    return (srcp.reshape(2, 163840 // 64, 64),
            dstp.reshape(2, 163840 // 64, 64))


def _score_tables_l0(a):
    ap = _pad_rows(a, NEG)
    return ap.reshape(RT, 2, 2).transpose(1, 0, 2).reshape(2, RT * 2)


def _score_tables_l1(a):
    ap = _pad_rows(a[:, None], NEG).reshape(RT)
    return jnp.broadcast_to(ap[None], (2, RT))


def _hs_table_l0(hs):
    hp = _pad_rows(hs)
    return jnp.concatenate([hp[:, :128], hp[:, 128:]], axis=0)


def _hs_table_l1(hs):
    hp = _pad_rows(hs)
    return jnp.concatenate([hp, hp], axis=0)


def _fold_l0(Wsrc, avec):
    return jnp.einsum('khc,hc->kh', Wsrc.reshape(64, 4, 64), avec)


def kernel(x_user, x_item, edge_index_u2i, edge_index_i2u,
           W_in_user, b_in_user, W_in_item, b_in_item,
           l0_u2i_Wsrc, l0_u2i_Wdst, l0_u2i_asrc, l0_u2i_adst, l0_u2i_b,
           l0_i2u_Wsrc, l0_i2u_Wdst, l0_i2u_asrc, l0_i2u_adst, l0_i2u_b,
           l1_u2i_Wsrc, l1_u2i_Wdst, l1_u2i_asrc, l1_u2i_adst, l1_u2i_b,
           l1_i2u_Wsrc, l1_i2u_Wdst, l1_i2u_asrc, l1_i2u_adst, l1_i2u_b):
    ws0_u2i = _fold_l0(l0_u2i_Wsrc, l0_u2i_asrc)
    wd0_u2i = _fold_l0(l0_u2i_Wdst, l0_u2i_adst)
    ws0_i2u = _fold_l0(l0_i2u_Wsrc, l0_i2u_asrc)
    wd0_i2u = _fold_l0(l0_i2u_Wdst, l0_i2u_adst)
    WbigU = jnp.concatenate([l0_u2i_Wsrc, ws0_u2i, wd0_i2u], axis=1)
    WbigI = jnp.concatenate([l0_i2u_Wsrc, ws0_i2u, wd0_u2i], axis=1)

    bigU = _tc_in_proj(x_user, W_in_user, b_in_user[None], WbigU)
    bigI = _tc_in_proj(x_item, W_in_item, b_in_item[None], WbigI)

    su0, du0 = _prep_edges_l0(edge_index_u2i)
    si0, di0 = _prep_edges_l0(edge_index_i2u)

    outU = _sc_conv_l0(_hs_table_l0(bigU[:, :256]),
                       _score_tables_l0(bigU[:, 256:260]),
                       _score_tables_l0(bigI[:, 260:264]),
                       su0, du0)
    outI = _sc_conv_l0(_hs_table_l0(bigI[:, :256]),
                       _score_tables_l0(bigI[:, 256:260]),
                       _score_tables_l0(bigU[:, 260:264]),
                       si0, di0)

    num_i = jnp.concatenate([outU[0, :N, :128], outU[1, :N, :128]], axis=1)
    den_i = jnp.concatenate([outU[0, :N, 128:130], outU[1, :N, 128:130]], axis=1)
    num_u = jnp.concatenate([outI[0, :N, :128], outI[1, :N, :128]], axis=1)
    den_u = jnp.concatenate([outI[0, :N, 128:130], outI[1, :N, 128:130]], axis=1)

    ws1_u2i = l1_u2i_Wsrc @ l1_u2i_asrc[0]
    wd1_u2i = l1_u2i_Wdst @ l1_u2i_adst[0]
    ws1_i2u = l1_i2u_Wsrc @ l1_i2u_asrc[0]
    wd1_i2u = l1_i2u_Wdst @ l1_i2u_adst[0]
    W1bigU = jnp.concatenate(
        [l1_u2i_Wsrc, ws1_u2i[:, None], wd1_i2u[:, None]], axis=1)
    W1bigI = jnp.concatenate(
        [l1_i2u_Wsrc, ws1_i2u[:, None], wd1_u2i[:, None]], axis=1)

    big1U = _tc_mid(num_u, den_u, l0_i2u_b[None], W1bigU)
    big1I = _tc_mid(num_i, den_i, l0_u2i_b[None], W1bigI)

    su1, du1 = _prep_edges_l1(edge_index_u2i)
    si1, di1 = _prep_edges_l1(edge_index_i2u)

    out1U = _sc_conv_l1(_hs_table_l1(big1U[:, :32]),
                        _score_tables_l1(big1U[:, 32]),
                        _score_tables_l1(big1I[:, 33]),
                        su1, du1)
    out1I = _sc_conv_l1(_hs_table_l1(big1I[:, :32]),
                        _score_tables_l1(big1I[:, 32]),
                        _score_tables_l1(big1U[:, 33]),
                        si1, di1)

    hi2 = _tc_final(out1U[0, :N, :33], out1U[1, :N, :33], l1_u2i_b[None])
    hu2 = _tc_final(out1I[0, :N, :33], out1I[1, :N, :33], l1_i2u_b[None])
    return hu2, hi2
